# speedup vs baseline: 1.0049x; 1.0049x over previous
import functools

import jax
import jax.numpy as jnp
from jax import lax
from jax.experimental import pallas as pl
from jax.experimental.pallas import tpu as pltpu

N = 10000
E = 320000
H = 128

BN = 400
BE = 4000


def _silu(v):
    return v * jax.nn.sigmoid(v)


def _prep_body(h_ref, at_ref, bt_ref, p_ref, q_ref):
    hb = h_ref[...]
    p_ref[...] = jnp.dot(hb, at_ref[...], preferred_element_type=jnp.float32)
    q_ref[...] = jnp.dot(hb, bt_ref[...], preferred_element_type=jnp.float32)


def _node_prep(h, At, Bt):
    return pl.pallas_call(
        _prep_body,
        grid=(N // BN,),
        in_specs=[
            pl.BlockSpec((BN, H), lambda i: (i, 0)),
            pl.BlockSpec((H, H), lambda i: (0, 0)),
            pl.BlockSpec((H, H), lambda i: (0, 0)),
        ],
        out_specs=[
            pl.BlockSpec((BN, H), lambda i: (i, 0)),
            pl.BlockSpec((BN, H), lambda i: (i, 0)),
        ],
        out_shape=[
            jax.ShapeDtypeStruct((N, H), jnp.float32),
            jax.ShapeDtypeStruct((N, H), jnp.float32),
        ],
    )(h, At, Bt)


def _edge_body(g1_ref, g2_ref, xs_ref, xd_ref, c2_ref, be1_ref, we2_ref,
               be2_ref, wc1_ref, bc1_ref, wc2_ref, m_ref, wd_ref):
    ds = xs_ref[...] - xd_ref[...]
    pre = (g1_ref[...] + g2_ref[...] + be1_ref[...]
           + jnp.dot(ds * ds, c2_ref[...], preferred_element_type=jnp.float32))
    m1 = _silu(pre)
    m = _silu(jnp.dot(m1, we2_ref[...], preferred_element_type=jnp.float32)
              + be2_ref[...])
    m_ref[...] = m
    c = _silu(jnp.dot(m, wc1_ref[...], preferred_element_type=jnp.float32)
              + bc1_ref[...])
    cw = jnp.dot(c, wc2_ref[...], preferred_element_type=jnp.float32)
    one3 = (lax.broadcasted_iota(jnp.int32, ds.shape, 1) == 3).astype(jnp.float32)
    wd_ref[...] = ds * cw + one3


def _edge_mlp(g1, g2, xs, xd, C2, b_e1, We2T, b_e2, Wc1T, b_c1, Wc2T):
    full = lambda r, c: pl.BlockSpec((r, c), lambda i: (0, 0))
    return pl.pallas_call(
        _edge_body,
        grid=(E // BE,),
        in_specs=[
            pl.BlockSpec((BE, H), lambda i: (i, 0)),
            pl.BlockSpec((BE, H), lambda i: (i, 0)),
            pl.BlockSpec((BE, 4), lambda i: (i, 0)),
            pl.BlockSpec((BE, 4), lambda i: (i, 0)),
            full(4, H), full(1, H), full(H, H), full(1, H),
            full(H, H), full(1, H), full(H, 1),
        ],
        out_specs=[
            pl.BlockSpec((BE, H), lambda i: (i, 0)),
            pl.BlockSpec((BE, 4), lambda i: (i, 0)),
        ],
        out_shape=[
            jax.ShapeDtypeStruct((E, H), jnp.float32),
            jax.ShapeDtypeStruct((E, 4), jnp.float32),
        ],
    )(g1, g2, xs, xd, C2, b_e1, We2T, b_e2, Wc1T, b_c1, Wc2T)


def _node_body(h_ref, x4_ref, agg0_ref, agg1_ref, cg0_ref, cg1_ref,
               n1a_ref, n1b_ref, bn1_ref, n2_ref, bn2_ref, lnw_ref, lnb_ref,
               ho_ref, xo_ref):
    hb = h_ref[...]
    cg = cg0_ref[...] + cg1_ref[...]
    cnt = jnp.maximum(cg[:, 3:4], 1.0)
    agg = (agg0_ref[...] + agg1_ref[...]) / cnt
    u = _silu(jnp.dot(hb, n1a_ref[...], preferred_element_type=jnp.float32)
              + jnp.dot(agg, n1b_ref[...], preferred_element_type=jnp.float32)
              + bn1_ref[...])
    t = jnp.dot(u, n2_ref[...], preferred_element_type=jnp.float32) + bn2_ref[...]
    ho = hb + t
    mu = jnp.mean(ho, axis=-1, keepdims=True)
    d = ho - mu
    var = jnp.mean(d * d, axis=-1, keepdims=True)
    ho_ref[...] = d * lax.rsqrt(var + 1e-5) * lnw_ref[...] + lnb_ref[...]
    xo_ref[...] = x4_ref[...] + cg / cnt


def _node_update(h, x4, agg0, agg1, cg0, cg1, N1aT, N1bT, b_n1, Wn2T, b_n2,
                 ln_w, ln_b):
    full = lambda r, c: pl.BlockSpec((r, c), lambda i: (0, 0))
    blk = lambda c: pl.BlockSpec((BN, c), lambda i: (i, 0))
    return pl.pallas_call(
        _node_body,
        grid=(N // BN,),
        in_specs=[
            blk(H), blk(4), blk(H), blk(H), blk(4), blk(4),
            full(H, H), full(H, H), full(1, H), full(H, H), full(1, H),
            full(1, H), full(1, H),
        ],
        out_specs=[blk(H), blk(4)],
        out_shape=[
            jax.ShapeDtypeStruct((N, H), jnp.float32),
            jax.ShapeDtypeStruct((N, 4), jnp.float32),
        ],
    )(h, x4, agg0, agg1, cg0, cg1, N1aT, N1bT, b_n1, Wn2T, b_n2, ln_w, ln_b)


def kernel(h, x, edge_index, W_e1, b_e1, W_e2, b_e2, W_n1, b_n1, W_n2, b_n2,
           W_c1, b_c1, W_c2, ln_w, ln_b):
    src = edge_index[0]
    dst = edge_index[1]
    x4 = jnp.pad(x, ((0, 0), (0, 1)))

    At = W_e1[:, :H].T
    Bt = W_e1[:, H:2 * H].T
    w_r = W_e1[:, 2 * H]
    C2 = jnp.broadcast_to(w_r, (4, H))

    p, q = _node_prep(h, At, Bt)

    g1 = jnp.take(p, src, axis=0)
    g2 = jnp.take(q, dst, axis=0)
    xs = jnp.take(x4, src, axis=0)
    xd = jnp.take(x4, dst, axis=0)

    m, wd = _edge_mlp(g1, g2, xs, xd, C2, b_e1.reshape(1, H), W_e2.T,
                      b_e2.reshape(1, H), W_c1.T, b_c1.reshape(1, H), W_c2.T)

    agg0 = jnp.zeros((N, H), jnp.float32).at[dst].add(m)
    cg0 = jnp.zeros((N, 4), jnp.float32).at[dst].add(wd)
    agg1 = jnp.zeros((N, H), jnp.float32)
    cg1 = jnp.zeros((N, 4), jnp.float32)

    ho, xo = _node_update(h, x4, agg0, agg1, cg0, cg1,
                          W_n1[:, :H].T, W_n1[:, H:].T, b_n1.reshape(1, H),
                          W_n2.T, b_n2.reshape(1, H), ln_w.reshape(1, H),
                          ln_b.reshape(1, H))
    return (ho, xo[:, :3])


# speedup vs baseline: 2.3799x; 2.3684x over previous
import functools

import jax
import jax.numpy as jnp
from jax import lax
from jax.experimental import pallas as pl
from jax.experimental.pallas import tpu as pltpu
from jax.experimental.pallas import tpu_sc as plsc

N = 10000
E = 320000
H = 128

BN = 400
BE = 4000

NC = 2
NS = 16
NW = NC * NS
GC = 128
CHUNKS = E // GC
FULL_ROUNDS = CHUNKS // NW
TAIL = CHUNKS - FULL_ROUNDS * NW

_SC_MESH = dict(core_axis_name="c", subcore_axis_name="s",
                num_cores=NC, num_subcores=NS)


def _gather_chunk(chunk, refs):
    (p_hbm, q_hbm, si_hbm, di_hbm, g1_hbm, g2_hbm, d4_hbm,
     si_v, di_v, g1_v, g2_v, d4_v, x4t_v, sem) = refs
    base = chunk * GC
    pltpu.sync_copy(si_hbm.at[pl.ds(base, GC)], si_v)
    pltpu.sync_copy(di_hbm.at[pl.ds(base, GC)], di_v)
    cp1 = pltpu.async_copy(p_hbm.at[si_v], g1_v, sem)
    cp2 = pltpu.async_copy(q_hbm.at[di_v], g2_v, sem)
    lanes = lax.iota(jnp.int32, 16)
    for k in range(GC // 16):
        rows4 = (lanes + (k * 16)) * 4
        si16 = si_v[pl.ds(k * 16, 16)]
        di16 = di_v[pl.ds(k * 16, 16)]
        dist = jnp.zeros((16,), jnp.float32)
        for cc in range(3):
            a = plsc.load_gather(x4t_v, [si16 + (cc * N)])
            b = plsc.load_gather(x4t_v, [di16 + (cc * N)])
            d = a - b
            dist = dist + d * d
            plsc.store_scatter(d4_v, [rows4 + cc], d)
        plsc.store_scatter(d4_v, [rows4 + 3], dist)
    cp1.wait()
    cp2.wait()
    pltpu.sync_copy(g1_v, g1_hbm.at[pl.ds(base, GC)])
    pltpu.sync_copy(g2_v, g2_hbm.at[pl.ds(base, GC)])
    pltpu.sync_copy(d4_v, d4_hbm.at[pl.ds(base * 4, GC * 4)])


@functools.partial(
    pl.kernel,
    out_type=[
        jax.ShapeDtypeStruct((E, H), jnp.float32),
        jax.ShapeDtypeStruct((E, H), jnp.float32),
        jax.ShapeDtypeStruct((E * 4,), jnp.float32),
    ],
    mesh=plsc.VectorSubcoreMesh(**_SC_MESH),
    scratch_types=[
        pltpu.VMEM((GC,), jnp.int32),
        pltpu.VMEM((GC,), jnp.int32),
        pltpu.VMEM((GC, H), jnp.float32),
        pltpu.VMEM((GC, H), jnp.float32),
        pltpu.VMEM((GC * 4,), jnp.float32),
        pltpu.VMEM((3 * N,), jnp.float32),
        pltpu.SemaphoreType.DMA,
    ],
    compiler_params=pltpu.CompilerParams(needs_layout_passes=False),
)
def _sc_gather(p_hbm, q_hbm, x4t_hbm, si_hbm, di_hbm,
               g1_hbm, g2_hbm, d4_hbm,
               si_v, di_v, g1_v, g2_v, d4_v, x4t_v, sem):
    refs = (p_hbm, q_hbm, si_hbm, di_hbm, g1_hbm, g2_hbm, d4_hbm,
            si_v, di_v, g1_v, g2_v, d4_v, x4t_v, sem)
    pltpu.sync_copy(x4t_hbm, x4t_v)
    wid = lax.axis_index("s") * NC + lax.axis_index("c")

    def body(r, _):
        _gather_chunk(r * NW + wid, refs)
        return 0

    lax.fori_loop(0, FULL_ROUNDS, body, 0)

    @pl.when(wid < TAIL)
    def _():
        _gather_chunk(FULL_ROUNDS * NW + wid, refs)


def _silu(v):
    return v * jax.nn.sigmoid(v)


def _prep_body(h_ref, at_ref, bt_ref, p_ref, q_ref):
    hb = h_ref[...]
    p_ref[...] = jnp.dot(hb, at_ref[...], preferred_element_type=jnp.float32)
    q_ref[...] = jnp.dot(hb, bt_ref[...], preferred_element_type=jnp.float32)


def _node_prep(h, At, Bt):
    return pl.pallas_call(
        _prep_body,
        grid=(N // BN,),
        in_specs=[
            pl.BlockSpec((BN, H), lambda i: (i, 0)),
            pl.BlockSpec((H, H), lambda i: (0, 0)),
            pl.BlockSpec((H, H), lambda i: (0, 0)),
        ],
        out_specs=[
            pl.BlockSpec((BN, H), lambda i: (i, 0)),
            pl.BlockSpec((BN, H), lambda i: (i, 0)),
        ],
        out_shape=[
            jax.ShapeDtypeStruct((N, H), jnp.float32),
            jax.ShapeDtypeStruct((N, H), jnp.float32),
        ],
    )(h, At, Bt)


def _edge_body(g1_ref, g2_ref, d4_ref, wr_ref, be1_ref, we2_ref,
               be2_ref, wc1_ref, bc1_ref, wc2_ref, m_ref, wd_ref):
    d4 = d4_ref[...]
    dist = d4[:, 3:4]
    pre = g1_ref[...] + g2_ref[...] + be1_ref[...] + dist * wr_ref[...]
    m1 = _silu(pre)
    m = _silu(jnp.dot(m1, we2_ref[...], preferred_element_type=jnp.float32)
              + be2_ref[...])
    m_ref[...] = m
    c = _silu(jnp.dot(m, wc1_ref[...], preferred_element_type=jnp.float32)
              + bc1_ref[...])
    cw = jnp.dot(c, wc2_ref[...], preferred_element_type=jnp.float32)
    is3 = lax.broadcasted_iota(jnp.int32, d4.shape, 1) == 3
    wd_ref[...] = jnp.where(is3, 1.0, d4 * cw)


def _edge_mlp(g1, g2, d4, wr, b_e1, We2T, b_e2, Wc1T, b_c1, Wc2T):
    full = lambda r, c: pl.BlockSpec((r, c), lambda i: (0, 0))
    return pl.pallas_call(
        _edge_body,
        grid=(E // BE,),
        in_specs=[
            pl.BlockSpec((BE, H), lambda i: (i, 0)),
            pl.BlockSpec((BE, H), lambda i: (i, 0)),
            pl.BlockSpec((BE, 4), lambda i: (i, 0)),
            full(1, H), full(1, H), full(H, H), full(1, H),
            full(H, H), full(1, H), full(H, 1),
        ],
        out_specs=[
            pl.BlockSpec((BE, H), lambda i: (i, 0)),
            pl.BlockSpec((BE, 4), lambda i: (i, 0)),
        ],
        out_shape=[
            jax.ShapeDtypeStruct((E, H), jnp.float32),
            jax.ShapeDtypeStruct((E, 4), jnp.float32),
        ],
    )(g1, g2, d4, wr, b_e1, We2T, b_e2, Wc1T, b_c1, Wc2T)


def _node_body(h_ref, x4_ref, agg0_ref, agg1_ref, cg0_ref, cg1_ref,
               n1a_ref, n1b_ref, bn1_ref, n2_ref, bn2_ref, lnw_ref, lnb_ref,
               ho_ref, xo_ref):
    hb = h_ref[...]
    cg = cg0_ref[...] + cg1_ref[...]
    cnt = jnp.maximum(cg[:, 3:4], 1.0)
    agg = (agg0_ref[...] + agg1_ref[...]) / cnt
    u = _silu(jnp.dot(hb, n1a_ref[...], preferred_element_type=jnp.float32)
              + jnp.dot(agg, n1b_ref[...], preferred_element_type=jnp.float32)
              + bn1_ref[...])
    t = jnp.dot(u, n2_ref[...], preferred_element_type=jnp.float32) + bn2_ref[...]
    ho = hb + t
    mu = jnp.mean(ho, axis=-1, keepdims=True)
    d = ho - mu
    var = jnp.mean(d * d, axis=-1, keepdims=True)
    ho_ref[...] = d * lax.rsqrt(var + 1e-5) * lnw_ref[...] + lnb_ref[...]
    xo_ref[...] = x4_ref[...] + cg / cnt


def _node_update(h, x4, agg0, agg1, cg0, cg1, N1aT, N1bT, b_n1, Wn2T, b_n2,
                 ln_w, ln_b):
    full = lambda r, c: pl.BlockSpec((r, c), lambda i: (0, 0))
    blk = lambda c: pl.BlockSpec((BN, c), lambda i: (i, 0))
    return pl.pallas_call(
        _node_body,
        grid=(N // BN,),
        in_specs=[
            blk(H), blk(4), blk(H), blk(H), blk(4), blk(4),
            full(H, H), full(H, H), full(1, H), full(H, H), full(1, H),
            full(1, H), full(1, H),
        ],
        out_specs=[blk(H), blk(4)],
        out_shape=[
            jax.ShapeDtypeStruct((N, H), jnp.float32),
            jax.ShapeDtypeStruct((N, 4), jnp.float32),
        ],
    )(h, x4, agg0, agg1, cg0, cg1, N1aT, N1bT, b_n1, Wn2T, b_n2, ln_w, ln_b)


def kernel(h, x, edge_index, W_e1, b_e1, W_e2, b_e2, W_n1, b_n1, W_n2, b_n2,
           W_c1, b_c1, W_c2, ln_w, ln_b):
    src = edge_index[0]
    dst = edge_index[1]
    x4 = jnp.pad(x, ((0, 0), (0, 1)))

    At = W_e1[:, :H].T
    Bt = W_e1[:, H:2 * H].T
    wr = W_e1[:, 2 * H].reshape(1, H)

    p, q = _node_prep(h, At, Bt)

    g1, g2, d4f = _sc_gather(p, q, x.T.reshape(3 * N), src, dst)
    d4 = d4f.reshape(E, 4)

    m, wd = _edge_mlp(g1, g2, d4, wr, b_e1.reshape(1, H), W_e2.T,
                      b_e2.reshape(1, H), W_c1.T, b_c1.reshape(1, H), W_c2.T)

    agg0 = jnp.zeros((N, H), jnp.float32).at[dst].add(m)
    cg0 = jnp.zeros((N, 4), jnp.float32).at[dst].add(wd)
    agg1 = jnp.zeros((N, H), jnp.float32)
    cg1 = jnp.zeros((N, 4), jnp.float32)

    ho, xo = _node_update(h, x4, agg0, agg1, cg0, cg1,
                          W_n1[:, :H].T, W_n1[:, H:].T, b_n1.reshape(1, H),
                          W_n2.T, b_n2.reshape(1, H), ln_w.reshape(1, H),
                          ln_b.reshape(1, H))
    return (ho, xo[:, :3])


# speedup vs baseline: 3.7788x; 1.5878x over previous
import functools

import jax
import jax.numpy as jnp
from jax import lax
from jax.experimental import pallas as pl
from jax.experimental.pallas import tpu as pltpu
from jax.experimental.pallas import tpu_sc as plsc

N = 10000
E = 320000
H = 128

BN = 400
BE = 4000

NC = 2
NS = 16
NW = NC * NS
GC = 128
CHUNKS = E // GC
FULL_ROUNDS = CHUNKS // NW
TAIL = CHUNKS - FULL_ROUNDS * NW

_SC_MESH = dict(core_axis_name="c", subcore_axis_name="s",
                num_cores=NC, num_subcores=NS)


def _gather_chunk(chunk, refs):
    (p_hbm, q_hbm, si_hbm, di_hbm, g1_hbm, g2_hbm, d4_hbm,
     si_v, di_v, g1_v, g2_v, d4_v, x4t_v, sem) = refs
    base = chunk * GC
    pltpu.sync_copy(si_hbm.at[pl.ds(base, GC)], si_v)
    pltpu.sync_copy(di_hbm.at[pl.ds(base, GC)], di_v)
    cp1 = pltpu.async_copy(p_hbm.at[si_v], g1_v, sem)
    cp2 = pltpu.async_copy(q_hbm.at[di_v], g2_v, sem)
    lanes = lax.iota(jnp.int32, 16)
    for k in range(GC // 16):
        rows4 = (lanes + (k * 16)) * 4
        si16 = si_v[pl.ds(k * 16, 16)]
        di16 = di_v[pl.ds(k * 16, 16)]
        dist = jnp.zeros((16,), jnp.float32)
        for cc in range(3):
            a = plsc.load_gather(x4t_v, [si16 + (cc * N)])
            b = plsc.load_gather(x4t_v, [di16 + (cc * N)])
            d = a - b
            dist = dist + d * d
            plsc.store_scatter(d4_v, [rows4 + cc], d)
        plsc.store_scatter(d4_v, [rows4 + 3], dist)
    cp1.wait()
    cp2.wait()
    pltpu.sync_copy(g1_v, g1_hbm.at[pl.ds(base, GC)])
    pltpu.sync_copy(g2_v, g2_hbm.at[pl.ds(base, GC)])
    pltpu.sync_copy(d4_v, d4_hbm.at[pl.ds(base * 4, GC * 4)])


@functools.partial(
    pl.kernel,
    out_type=[
        jax.ShapeDtypeStruct((E, H), jnp.float32),
        jax.ShapeDtypeStruct((E, H), jnp.float32),
        jax.ShapeDtypeStruct((E * 4,), jnp.float32),
    ],
    mesh=plsc.VectorSubcoreMesh(**_SC_MESH),
    scratch_types=[
        pltpu.VMEM((GC,), jnp.int32),
        pltpu.VMEM((GC,), jnp.int32),
        pltpu.VMEM((GC, H), jnp.float32),
        pltpu.VMEM((GC, H), jnp.float32),
        pltpu.VMEM((GC * 4,), jnp.float32),
        pltpu.VMEM((3 * N,), jnp.float32),
        pltpu.SemaphoreType.DMA,
    ],
    compiler_params=pltpu.CompilerParams(needs_layout_passes=False),
)
def _sc_gather(p_hbm, q_hbm, x4t_hbm, si_hbm, di_hbm,
               g1_hbm, g2_hbm, d4_hbm,
               si_v, di_v, g1_v, g2_v, d4_v, x4t_v, sem):
    refs = (p_hbm, q_hbm, si_hbm, di_hbm, g1_hbm, g2_hbm, d4_hbm,
            si_v, di_v, g1_v, g2_v, d4_v, x4t_v, sem)
    pltpu.sync_copy(x4t_hbm, x4t_v)
    wid = lax.axis_index("s") * NC + lax.axis_index("c")

    def body(r, _):
        _gather_chunk(r * NW + wid, refs)
        return 0

    lax.fori_loop(0, FULL_ROUNDS, body, 0)

    @pl.when(wid < TAIL)
    def _():
        _gather_chunk(FULL_ROUNDS * NW + wid, refs)


def _silu(v):
    return v * jax.nn.sigmoid(v)


def _prep_body(h_ref, at_ref, bt_ref, p_ref, q_ref):
    hb = h_ref[...]
    p_ref[...] = jnp.dot(hb, at_ref[...], preferred_element_type=jnp.float32)
    q_ref[...] = jnp.dot(hb, bt_ref[...], preferred_element_type=jnp.float32)


def _node_prep(h, At, Bt):
    return pl.pallas_call(
        _prep_body,
        grid=(N // BN,),
        in_specs=[
            pl.BlockSpec((BN, H), lambda i: (i, 0)),
            pl.BlockSpec((H, H), lambda i: (0, 0)),
            pl.BlockSpec((H, H), lambda i: (0, 0)),
        ],
        out_specs=[
            pl.BlockSpec((BN, H), lambda i: (i, 0)),
            pl.BlockSpec((BN, H), lambda i: (i, 0)),
        ],
        out_shape=[
            jax.ShapeDtypeStruct((N, H), jnp.float32),
            jax.ShapeDtypeStruct((N, H), jnp.float32),
        ],
    )(h, At, Bt)


SC_CHUNKS = CHUNKS // NC
SC_ROUNDS = SC_CHUNKS // NS
SC_TAIL = SC_CHUNKS - SC_ROUNDS * NS
N2 = 10240
RT = N2 // NS
CR = N2 * 4 // H


def _scatter_chunk(chunk, refs):
    (m_hbm, di_hbm, di2_v, m_v, acc_sh, sem) = refs
    base = chunk * GC
    pltpu.sync_copy(di_hbm.at[pl.ds(base, GC)], di2_v.at[0])
    pltpu.async_copy(m_hbm.at[pl.ds(base, GC)], m_v, sem).wait()
    pltpu.sync_copy(m_v, acc_sh.at[di2_v.at[0]], add=True)


@functools.partial(
    pl.kernel,
    out_type=jax.ShapeDtypeStruct((NC * N2, H), jnp.float32),
    mesh=plsc.VectorSubcoreMesh(**_SC_MESH),
    scratch_types=[
        pltpu.VMEM((1, GC), jnp.int32),
        pltpu.VMEM((GC, H), jnp.float32),
        pltpu.VMEM_SHARED((N2, H), jnp.float32),
        pltpu.SemaphoreType.DMA,
    ],
    compiler_params=pltpu.CompilerParams(needs_layout_passes=False),
)
def _sc_scatter(m_hbm, di_hbm, aggp_hbm, di2_v, m_v, acc_sh, sem):
    c = lax.axis_index("c")
    s = lax.axis_index("s")
    refs = (m_hbm, di_hbm, di2_v, m_v, acc_sh, sem)

    def zrow(j, _):
        for k in range(H // 16):
            m_v[j, pl.ds(k * 16, 16)] = jnp.zeros((16,), jnp.float32)
        return 0

    lax.fori_loop(0, GC, zrow, 0)

    rows0 = s * RT
    for j in range(RT // GC):
        pltpu.sync_copy(m_v, acc_sh.at[pl.ds(rows0 + j * GC, GC)])
    plsc.subcore_barrier()

    def body(r, _):
        _scatter_chunk(c * SC_CHUNKS + r * NS + s, refs)
        return 0

    lax.fori_loop(0, SC_ROUNDS, body, 0)

    @pl.when(s < SC_TAIL)
    def _():
        _scatter_chunk(c * SC_CHUNKS + SC_ROUNDS * NS + s, refs)

    plsc.subcore_barrier()
    for j in range(RT // GC):
        r0 = rows0 + j * GC
        pltpu.sync_copy(acc_sh.at[pl.ds(r0, GC)], m_v)
        pltpu.sync_copy(m_v, aggp_hbm.at[pl.ds(c * N2 + r0, GC)])


def _cscatter_chunk(chunk, refs):
    (wd_hbm, di_hbm, di_v, wd_v, cacc_v) = refs
    base = chunk * GC
    pltpu.sync_copy(di_hbm.at[pl.ds(base, GC)], di_v)
    pltpu.sync_copy(wd_hbm.at[pl.ds(base * 4, GC * 4)], wd_v)
    lanes = lax.iota(jnp.int32, 16)
    for k in range(GC // 16):
        dstv = di_v[pl.ds(k * 16, 16)]
        src4 = (lanes + (k * 16)) * 4
        dst4 = dstv * 4
        for cc in range(4):
            val = plsc.load_gather(wd_v, [src4 + cc])
            tgt = dst4 + cc
            plsc.addupdate_scatter(
                cacc_v, [lax.shift_right_logical(tgt, 7),
                         lax.bitwise_and(tgt, 127)], val)


@functools.partial(
    pl.kernel,
    out_type=jax.ShapeDtypeStruct((NW * CR, H), jnp.float32),
    mesh=plsc.VectorSubcoreMesh(**_SC_MESH),
    scratch_types=[
        pltpu.VMEM((GC,), jnp.int32),
        pltpu.VMEM((GC * 4,), jnp.float32),
        pltpu.VMEM((CR, H), jnp.float32),
    ],
    compiler_params=pltpu.CompilerParams(needs_layout_passes=False),
)
def _sc_cscatter(wd_hbm, di_hbm, caccp_hbm, di_v, wd_v, cacc_v):
    c = lax.axis_index("c")
    s = lax.axis_index("s")
    wid = s * NC + c
    refs = (wd_hbm, di_hbm, di_v, wd_v, cacc_v)

    def zc(j, _):
        for k in range(H // 16):
            cacc_v[j, pl.ds(k * 16, 16)] = jnp.zeros((16,), jnp.float32)
        return 0

    lax.fori_loop(0, CR, zc, 0)

    def body(r, _):
        _cscatter_chunk(r * NW + wid, refs)
        return 0

    lax.fori_loop(0, FULL_ROUNDS, body, 0)

    @pl.when(wid < TAIL)
    def _():
        _cscatter_chunk(FULL_ROUNDS * NW + wid, refs)

    pltpu.sync_copy(cacc_v, caccp_hbm.at[pl.ds(wid * CR, CR)])


def _edge_body(g1_ref, g2_ref, d4_ref, wr_ref, be1_ref, we2_ref,
               be2_ref, wc1_ref, bc1_ref, wc2_ref, m_ref, wd_ref):
    d4 = d4_ref[...]
    dist = d4[:, 3:4]
    pre = g1_ref[...] + g2_ref[...] + be1_ref[...] + dist * wr_ref[...]
    m1 = _silu(pre)
    m = _silu(jnp.dot(m1, we2_ref[...], preferred_element_type=jnp.float32)
              + be2_ref[...])
    m_ref[...] = m
    c = _silu(jnp.dot(m, wc1_ref[...], preferred_element_type=jnp.float32)
              + bc1_ref[...])
    cw = jnp.dot(c, wc2_ref[...], preferred_element_type=jnp.float32)
    is3 = lax.broadcasted_iota(jnp.int32, d4.shape, 1) == 3
    wd_ref[...] = jnp.where(is3, 1.0, d4 * cw)


def _edge_mlp(g1, g2, d4, wr, b_e1, We2T, b_e2, Wc1T, b_c1, Wc2T):
    full = lambda r, c: pl.BlockSpec((r, c), lambda i: (0, 0))
    return pl.pallas_call(
        _edge_body,
        grid=(E // BE,),
        in_specs=[
            pl.BlockSpec((BE, H), lambda i: (i, 0)),
            pl.BlockSpec((BE, H), lambda i: (i, 0)),
            pl.BlockSpec((BE, 4), lambda i: (i, 0)),
            full(1, H), full(1, H), full(H, H), full(1, H),
            full(H, H), full(1, H), full(H, 1),
        ],
        out_specs=[
            pl.BlockSpec((BE, H), lambda i: (i, 0)),
            pl.BlockSpec((BE, 4), lambda i: (i, 0)),
        ],
        out_shape=[
            jax.ShapeDtypeStruct((E, H), jnp.float32),
            jax.ShapeDtypeStruct((E, 4), jnp.float32),
        ],
    )(g1, g2, d4, wr, b_e1, We2T, b_e2, Wc1T, b_c1, Wc2T)


def _node_body(h_ref, x4_ref, agg0_ref, agg1_ref, cgp_ref,
               n1a_ref, n1b_ref, bn1_ref, n2_ref, bn2_ref, lnw_ref, lnb_ref,
               ho_ref, xo_ref):
    hb = h_ref[...]
    cg = jnp.sum(cgp_ref[...], axis=0)
    cnt = jnp.maximum(cg[:, 3:4], 1.0)
    agg = (agg0_ref[...] + agg1_ref[...]) / cnt
    u = _silu(jnp.dot(hb, n1a_ref[...], preferred_element_type=jnp.float32)
              + jnp.dot(agg, n1b_ref[...], preferred_element_type=jnp.float32)
              + bn1_ref[...])
    t = jnp.dot(u, n2_ref[...], preferred_element_type=jnp.float32) + bn2_ref[...]
    ho = hb + t
    mu = jnp.mean(ho, axis=-1, keepdims=True)
    d = ho - mu
    var = jnp.mean(d * d, axis=-1, keepdims=True)
    ho_ref[...] = d * lax.rsqrt(var + 1e-5) * lnw_ref[...] + lnb_ref[...]
    xo_ref[...] = x4_ref[...] + cg / cnt


def _node_update(h, x4, agg0, agg1, cgp, N1aT, N1bT, b_n1, Wn2T, b_n2,
                 ln_w, ln_b):
    full = lambda r, c: pl.BlockSpec((r, c), lambda i: (0, 0))
    blk = lambda c: pl.BlockSpec((BN, c), lambda i: (i, 0))
    return pl.pallas_call(
        _node_body,
        grid=(N // BN,),
        in_specs=[
            blk(H), blk(4), blk(H), blk(H),
            pl.BlockSpec((NW, BN, 4), lambda i: (0, i, 0)),
            full(H, H), full(H, H), full(1, H), full(H, H), full(1, H),
            full(1, H), full(1, H),
        ],
        out_specs=[blk(H), blk(4)],
        out_shape=[
            jax.ShapeDtypeStruct((N, H), jnp.float32),
            jax.ShapeDtypeStruct((N, 4), jnp.float32),
        ],
    )(h, x4, agg0, agg1, cgp, N1aT, N1bT, b_n1, Wn2T, b_n2, ln_w, ln_b)


def kernel(h, x, edge_index, W_e1, b_e1, W_e2, b_e2, W_n1, b_n1, W_n2, b_n2,
           W_c1, b_c1, W_c2, ln_w, ln_b):
    src = edge_index[0]
    dst = edge_index[1]
    x4 = jnp.pad(x, ((0, 0), (0, 1)))

    At = W_e1[:, :H].T
    Bt = W_e1[:, H:2 * H].T
    wr = W_e1[:, 2 * H].reshape(1, H)

    p, q = _node_prep(h, At, Bt)

    g1, g2, d4f = _sc_gather(p, q, x.T.reshape(3 * N), src, dst)
    d4 = d4f.reshape(E, 4)

    m, wd = _edge_mlp(g1, g2, d4, wr, b_e1.reshape(1, H), W_e2.T,
                      b_e2.reshape(1, H), W_c1.T, b_c1.reshape(1, H), W_c2.T)

    aggp = _sc_scatter(m, dst)
    caccp = _sc_cscatter(wd.reshape(E * 4), dst)
    cgp = caccp.reshape(NW, N2, 4)
    agg0 = aggp[:N2]
    agg1 = aggp[N2:]

    ho, xo = _node_update(h, x4, agg0, agg1, cgp,
                          W_n1[:, :H].T, W_n1[:, H:].T, b_n1.reshape(1, H),
                          W_n2.T, b_n2.reshape(1, H), ln_w.reshape(1, H),
                          ln_b.reshape(1, H))
    return (ho, xo[:, :3])


# speedup vs baseline: 4.4098x; 1.1670x over previous
import functools

import jax
import jax.numpy as jnp
from jax import lax
from jax.experimental import pallas as pl
from jax.experimental.pallas import tpu as pltpu
from jax.experimental.pallas import tpu_sc as plsc

N = 10000
E = 320000
H = 128

BN = 400
BE = 4000

NC = 2
NS = 16
NW = NC * NS
GC = 128
CHUNKS = E // GC
FULL_ROUNDS = CHUNKS // NW
TAIL = CHUNKS - FULL_ROUNDS * NW

_SC_MESH = dict(core_axis_name="c", subcore_axis_name="s",
                num_cores=NC, num_subcores=NS)


def _g_compute_d4(si_vb, di_vb, d4_vb, x4t_v):
    lanes = lax.iota(jnp.int32, 16)
    for k in range(GC // 16):
        rows4 = (lanes + (k * 16)) * 4
        si16 = si_vb[pl.ds(k * 16, 16)]
        di16 = di_vb[pl.ds(k * 16, 16)]
        dist = jnp.zeros((16,), jnp.float32)
        for cc in range(3):
            a = plsc.load_gather(x4t_v, [si16 + (cc * N)])
            b2 = plsc.load_gather(x4t_v, [di16 + (cc * N)])
            d = a - b2
            dist = dist + d * d
            plsc.store_scatter(d4_vb, [rows4 + cc], d)
        plsc.store_scatter(d4_vb, [rows4 + 3], dist)


@functools.partial(
    pl.kernel,
    out_type=[
        jax.ShapeDtypeStruct((E, H), jnp.float32),
        jax.ShapeDtypeStruct((E, H), jnp.float32),
        jax.ShapeDtypeStruct((E * 4,), jnp.float32),
    ],
    mesh=plsc.VectorSubcoreMesh(**_SC_MESH),
    scratch_types=[
        pltpu.VMEM((GC,), jnp.int32),
        pltpu.VMEM((GC,), jnp.int32),
        pltpu.VMEM((GC,), jnp.int32),
        pltpu.VMEM((GC,), jnp.int32),
        pltpu.VMEM((GC, H), jnp.float32),
        pltpu.VMEM((GC, H), jnp.float32),
        pltpu.VMEM((GC, H), jnp.float32),
        pltpu.VMEM((GC, H), jnp.float32),
        pltpu.VMEM((GC * 4,), jnp.float32),
        pltpu.VMEM((GC * 4,), jnp.float32),
        pltpu.VMEM((3 * N,), jnp.float32),
        pltpu.SemaphoreType.DMA,
        pltpu.SemaphoreType.DMA,
        pltpu.SemaphoreType.DMA,
        pltpu.SemaphoreType.DMA,
        pltpu.SemaphoreType.DMA,
        pltpu.SemaphoreType.DMA,
    ],
    compiler_params=pltpu.CompilerParams(needs_layout_passes=False),
)
def _sc_gather(p_hbm, q_hbm, x4t_hbm, si_hbm, di_hbm,
               g1_hbm, g2_hbm, d4_hbm,
               siA, siB, diA, diB, g1A, g1B, g2A, g2B, d4A, d4B, x4t_v,
               smi0, smi1, smg0, smg1, smw0, smw1):
    pltpu.sync_copy(x4t_hbm, x4t_v)
    wid = lax.axis_index("s") * NC + lax.axis_index("c")
    si_v = (siA, siB)
    di_v = (diA, diB)
    g1_v = (g1A, g1B)
    g2_v = (g2A, g2B)
    d4_v = (d4A, d4B)
    sem_i = (smi0, smi1)
    sem_g = (smg0, smg1)
    sem_w = (smw0, smw1)

    def idx_load(r, b):
        base = (r * NW + wid) * GC
        pltpu.async_copy(si_hbm.at[pl.ds(base, GC)], si_v[b], sem_i[b])
        pltpu.async_copy(di_hbm.at[pl.ds(base, GC)], di_v[b], sem_i[b])

    def idx_drain(b):
        pltpu.make_async_copy(si_hbm.at[pl.ds(0, GC)], si_v[b],
                              sem_i[b]).wait()
        pltpu.make_async_copy(di_hbm.at[pl.ds(0, GC)], di_v[b],
                              sem_i[b]).wait()

    def write_drain(b):
        pltpu.make_async_copy(g1_v[b], g1_hbm.at[pl.ds(0, GC)],
                              sem_w[b]).wait()
        pltpu.make_async_copy(g2_v[b], g2_hbm.at[pl.ds(0, GC)],
                              sem_w[b]).wait()
        pltpu.make_async_copy(d4_v[b], d4_hbm.at[pl.ds(0, GC * 4)],
                              sem_w[b]).wait()

    idx_load(0, 0)

    def body_one(r, b):
        base = (r * NW + wid) * GC

        @pl.when(r >= 2)
        def _():
            write_drain(b)

        idx_drain(b)
        cp1 = pltpu.async_copy(p_hbm.at[si_v[b]], g1_v[b], sem_g[b])
        cp2 = pltpu.async_copy(q_hbm.at[di_v[b]], g2_v[b], sem_g[b])

        @pl.when(r + 1 < FULL_ROUNDS)
        def _():
            idx_load(r + 1, 1 - b)

        _g_compute_d4(si_v[b], di_v[b], d4_v[b], x4t_v)
        cp1.wait()
        cp2.wait()
        pltpu.async_copy(g1_v[b], g1_hbm.at[pl.ds(base, GC)], sem_w[b])
        pltpu.async_copy(g2_v[b], g2_hbm.at[pl.ds(base, GC)], sem_w[b])
        pltpu.async_copy(d4_v[b], d4_hbm.at[pl.ds(base * 4, GC * 4)],
                         sem_w[b])

    def body(i, _):
        body_one(2 * i, 0)
        body_one(2 * i + 1, 1)
        return 0

    lax.fori_loop(0, FULL_ROUNDS // 2, body, 0)
    write_drain(0)
    write_drain(1)

    @pl.when(wid < TAIL)
    def _():
        base = (FULL_ROUNDS * NW + wid) * GC
        pltpu.sync_copy(si_hbm.at[pl.ds(base, GC)], siA)
        pltpu.sync_copy(di_hbm.at[pl.ds(base, GC)], diA)
        cp1 = pltpu.async_copy(p_hbm.at[siA], g1A, smg0)
        cp2 = pltpu.async_copy(q_hbm.at[diA], g2A, smg0)
        _g_compute_d4(siA, diA, d4A, x4t_v)
        cp1.wait()
        cp2.wait()
        pltpu.sync_copy(g1A, g1_hbm.at[pl.ds(base, GC)])
        pltpu.sync_copy(g2A, g2_hbm.at[pl.ds(base, GC)])
        pltpu.sync_copy(d4A, d4_hbm.at[pl.ds(base * 4, GC * 4)])


def _silu(v):
    return v * jax.nn.sigmoid(v)


def _prep_body(h_ref, at_ref, bt_ref, p_ref, q_ref):
    hb = h_ref[...]
    p_ref[...] = jnp.dot(hb, at_ref[...], preferred_element_type=jnp.float32)
    q_ref[...] = jnp.dot(hb, bt_ref[...], preferred_element_type=jnp.float32)


def _node_prep(h, At, Bt):
    return pl.pallas_call(
        _prep_body,
        grid=(N // BN,),
        in_specs=[
            pl.BlockSpec((BN, H), lambda i: (i, 0)),
            pl.BlockSpec((H, H), lambda i: (0, 0)),
            pl.BlockSpec((H, H), lambda i: (0, 0)),
        ],
        out_specs=[
            pl.BlockSpec((BN, H), lambda i: (i, 0)),
            pl.BlockSpec((BN, H), lambda i: (i, 0)),
        ],
        out_shape=[
            jax.ShapeDtypeStruct((N, H), jnp.float32),
            jax.ShapeDtypeStruct((N, H), jnp.float32),
        ],
    )(h, At, Bt)


SC_CHUNKS = CHUNKS // NC
SC_ROUNDS = SC_CHUNKS // NS
SC_TAIL = SC_CHUNKS - SC_ROUNDS * NS
N2 = 10240
RT = N2 // NS
CR = N2 * 4 // H


@functools.partial(
    pl.kernel,
    out_type=jax.ShapeDtypeStruct((NC * N2, H), jnp.float32),
    mesh=plsc.VectorSubcoreMesh(**_SC_MESH),
    scratch_types=[
        pltpu.VMEM((1, GC), jnp.int32),
        pltpu.VMEM((1, GC), jnp.int32),
        pltpu.VMEM((GC, H), jnp.float32),
        pltpu.VMEM((GC, H), jnp.float32),
        pltpu.VMEM_SHARED((N2, H), jnp.float32),
        pltpu.SemaphoreType.DMA,
        pltpu.SemaphoreType.DMA,
    ],
    compiler_params=pltpu.CompilerParams(needs_layout_passes=False),
)
def _sc_scatter(m_hbm, di_hbm, aggp_hbm, diW0, diW1, m0, m1, acc_sh,
                sr0, sr1):
    c = lax.axis_index("c")
    s = lax.axis_index("s")
    di_v = (diW0, diW1)
    m_v = (m0, m1)
    sem_r = (sr0, sr1)

    def zrow(j, _):
        for k in range(H // 16):
            m0[j, pl.ds(k * 16, 16)] = jnp.zeros((16,), jnp.float32)
        return 0

    lax.fori_loop(0, GC, zrow, 0)

    rows0 = s * RT
    for j in range(RT // GC):
        pltpu.sync_copy(m0, acc_sh.at[pl.ds(rows0 + j * GC, GC)])
    plsc.subcore_barrier()

    n_chunks = SC_ROUNDS + 1

    def chunk_id(r):
        return c * SC_CHUNKS + r * NS + s

    def read(r, b):
        base = chunk_id(r) * GC
        pltpu.async_copy(di_hbm.at[pl.ds(base, GC)], di_v[b].at[0], sem_r[b])
        pltpu.async_copy(m_hbm.at[pl.ds(base, GC)], m_v[b], sem_r[b])

    def read_drain(b):
        pltpu.make_async_copy(di_hbm.at[pl.ds(0, GC)], di_v[b].at[0],
                              sem_r[b]).wait()
        pltpu.make_async_copy(m_hbm.at[pl.ds(0, GC)], m_v[b],
                              sem_r[b]).wait()

    def live(r):
        return jnp.logical_or(r < SC_ROUNDS,
                              jnp.logical_and(r == SC_ROUNDS, s < SC_TAIL))

    @pl.when(live(0))
    def _():
        read(0, 0)

    def body_one(r, b):
        @pl.when(live(r + 1))
        def _():
            read(r + 1, 1 - b)

        @pl.when(live(r))
        def _():
            read_drain(b)
            pltpu.sync_copy(m_v[b], acc_sh.at[di_v[b].at[0]], add=True)

    def body(i, _):
        body_one(2 * i, 0)
        body_one(2 * i + 1, 1)
        return 0

    lax.fori_loop(0, n_chunks // 2, body, 0)
    body_one(n_chunks - 1, 0)

    plsc.subcore_barrier()
    for j in range(RT // GC):
        r0 = rows0 + j * GC
        pltpu.sync_copy(acc_sh.at[pl.ds(r0, GC)], m0)
        pltpu.sync_copy(m0, aggp_hbm.at[pl.ds(c * N2 + r0, GC)])


def _cscatter_chunk(chunk, refs):
    (wd_hbm, di_hbm, di_v, wd_v, cacc_v) = refs
    base = chunk * GC
    pltpu.sync_copy(di_hbm.at[pl.ds(base, GC)], di_v)
    pltpu.sync_copy(wd_hbm.at[pl.ds(base * 4, GC * 4)], wd_v)
    lanes = lax.iota(jnp.int32, 16)
    for k in range(GC // 16):
        dstv = di_v[pl.ds(k * 16, 16)]
        src4 = (lanes + (k * 16)) * 4
        dst4 = dstv * 4
        for cc in range(4):
            val = plsc.load_gather(wd_v, [src4 + cc])
            tgt = dst4 + cc
            plsc.addupdate_scatter(
                cacc_v, [lax.shift_right_logical(tgt, 7),
                         lax.bitwise_and(tgt, 127)], val)


@functools.partial(
    pl.kernel,
    out_type=jax.ShapeDtypeStruct((NW * CR, H), jnp.float32),
    mesh=plsc.VectorSubcoreMesh(**_SC_MESH),
    scratch_types=[
        pltpu.VMEM((GC,), jnp.int32),
        pltpu.VMEM((GC * 4,), jnp.float32),
        pltpu.VMEM((CR, H), jnp.float32),
    ],
    compiler_params=pltpu.CompilerParams(needs_layout_passes=False),
)
def _sc_cscatter(wd_hbm, di_hbm, caccp_hbm, di_v, wd_v, cacc_v):
    c = lax.axis_index("c")
    s = lax.axis_index("s")
    wid = s * NC + c
    refs = (wd_hbm, di_hbm, di_v, wd_v, cacc_v)

    def zc(j, _):
        for k in range(H // 16):
            cacc_v[j, pl.ds(k * 16, 16)] = jnp.zeros((16,), jnp.float32)
        return 0

    lax.fori_loop(0, CR, zc, 0)

    def body(r, _):
        _cscatter_chunk(r * NW + wid, refs)
        return 0

    lax.fori_loop(0, FULL_ROUNDS, body, 0)

    @pl.when(wid < TAIL)
    def _():
        _cscatter_chunk(FULL_ROUNDS * NW + wid, refs)

    pltpu.sync_copy(cacc_v, caccp_hbm.at[pl.ds(wid * CR, CR)])


def _edge_body(g1_ref, g2_ref, d4_ref, wr_ref, be1_ref, we2_ref,
               be2_ref, wc1_ref, bc1_ref, wc2_ref, m_ref, wd_ref):
    d4 = d4_ref[...]
    dist = d4[:, 3:4]
    pre = g1_ref[...] + g2_ref[...] + be1_ref[...] + dist * wr_ref[...]
    m1 = _silu(pre)
    m = _silu(jnp.dot(m1, we2_ref[...], preferred_element_type=jnp.float32)
              + be2_ref[...])
    m_ref[...] = m
    c = _silu(jnp.dot(m, wc1_ref[...], preferred_element_type=jnp.float32)
              + bc1_ref[...])
    cw = jnp.dot(c, wc2_ref[...], preferred_element_type=jnp.float32)
    is3 = lax.broadcasted_iota(jnp.int32, d4.shape, 1) == 3
    wd_ref[...] = jnp.where(is3, 1.0, d4 * cw)


def _edge_mlp(g1, g2, d4, wr, b_e1, We2T, b_e2, Wc1T, b_c1, Wc2T):
    full = lambda r, c: pl.BlockSpec((r, c), lambda i: (0, 0))
    return pl.pallas_call(
        _edge_body,
        grid=(E // BE,),
        in_specs=[
            pl.BlockSpec((BE, H), lambda i: (i, 0)),
            pl.BlockSpec((BE, H), lambda i: (i, 0)),
            pl.BlockSpec((BE, 4), lambda i: (i, 0)),
            full(1, H), full(1, H), full(H, H), full(1, H),
            full(H, H), full(1, H), full(H, 1),
        ],
        out_specs=[
            pl.BlockSpec((BE, H), lambda i: (i, 0)),
            pl.BlockSpec((BE, 4), lambda i: (i, 0)),
        ],
        out_shape=[
            jax.ShapeDtypeStruct((E, H), jnp.float32),
            jax.ShapeDtypeStruct((E, 4), jnp.float32),
        ],
    )(g1, g2, d4, wr, b_e1, We2T, b_e2, Wc1T, b_c1, Wc2T)


def _node_body(h_ref, x4_ref, agg0_ref, agg1_ref, cgp_ref,
               n1a_ref, n1b_ref, bn1_ref, n2_ref, bn2_ref, lnw_ref, lnb_ref,
               ho_ref, xo_ref):
    hb = h_ref[...]
    cg = jnp.sum(cgp_ref[...], axis=0)
    cnt = jnp.maximum(cg[:, 3:4], 1.0)
    agg = (agg0_ref[...] + agg1_ref[...]) / cnt
    u = _silu(jnp.dot(hb, n1a_ref[...], preferred_element_type=jnp.float32)
              + jnp.dot(agg, n1b_ref[...], preferred_element_type=jnp.float32)
              + bn1_ref[...])
    t = jnp.dot(u, n2_ref[...], preferred_element_type=jnp.float32) + bn2_ref[...]
    ho = hb + t
    mu = jnp.mean(ho, axis=-1, keepdims=True)
    d = ho - mu
    var = jnp.mean(d * d, axis=-1, keepdims=True)
    ho_ref[...] = d * lax.rsqrt(var + 1e-5) * lnw_ref[...] + lnb_ref[...]
    xo_ref[...] = x4_ref[...] + cg / cnt


def _node_update(h, x4, agg0, agg1, cgp, N1aT, N1bT, b_n1, Wn2T, b_n2,
                 ln_w, ln_b):
    full = lambda r, c: pl.BlockSpec((r, c), lambda i: (0, 0))
    blk = lambda c: pl.BlockSpec((BN, c), lambda i: (i, 0))
    return pl.pallas_call(
        _node_body,
        grid=(N // BN,),
        in_specs=[
            blk(H), blk(4), blk(H), blk(H),
            pl.BlockSpec((NW, BN, 4), lambda i: (0, i, 0)),
            full(H, H), full(H, H), full(1, H), full(H, H), full(1, H),
            full(1, H), full(1, H),
        ],
        out_specs=[blk(H), blk(4)],
        out_shape=[
            jax.ShapeDtypeStruct((N, H), jnp.float32),
            jax.ShapeDtypeStruct((N, 4), jnp.float32),
        ],
    )(h, x4, agg0, agg1, cgp, N1aT, N1bT, b_n1, Wn2T, b_n2, ln_w, ln_b)


def kernel(h, x, edge_index, W_e1, b_e1, W_e2, b_e2, W_n1, b_n1, W_n2, b_n2,
           W_c1, b_c1, W_c2, ln_w, ln_b):
    src = edge_index[0]
    dst = edge_index[1]
    x4 = jnp.pad(x, ((0, 0), (0, 1)))

    At = W_e1[:, :H].T
    Bt = W_e1[:, H:2 * H].T
    wr = W_e1[:, 2 * H].reshape(1, H)

    p, q = _node_prep(h, At, Bt)

    g1, g2, d4f = _sc_gather(p, q, x.T.reshape(3 * N), src, dst)
    d4 = d4f.reshape(E, 4)

    m, wd = _edge_mlp(g1, g2, d4, wr, b_e1.reshape(1, H), W_e2.T,
                      b_e2.reshape(1, H), W_c1.T, b_c1.reshape(1, H), W_c2.T)

    aggp = _sc_scatter(m, dst)
    caccp = _sc_cscatter(wd.reshape(E * 4), dst)
    cgp = caccp.reshape(NW, N2, 4)
    agg0 = aggp[:N2]
    agg1 = aggp[N2:]

    ho, xo = _node_update(h, x4, agg0, agg1, cgp,
                          W_n1[:, :H].T, W_n1[:, H:].T, b_n1.reshape(1, H),
                          W_n2.T, b_n2.reshape(1, H), ln_w.reshape(1, H),
                          ln_b.reshape(1, H))
    return (ho, xo[:, :3])


# speedup vs baseline: 7.2013x; 1.6330x over previous
import functools

import jax
import jax.numpy as jnp
from jax import lax
from jax.experimental import pallas as pl
from jax.experimental.pallas import tpu as pltpu
from jax.experimental.pallas import tpu_sc as plsc

N = 10000
E = 320000
H = 128

BN = 400
BN2 = 2048
BE = 2560

NC = 2
NS = 16
NW = NC * NS
GC = 128
CHUNKS = E // GC
FULL_ROUNDS = CHUNKS // NW
TAIL = CHUNKS - FULL_ROUNDS * NW

_SC_MESH = dict(core_axis_name="c", subcore_axis_name="s",
                num_cores=NC, num_subcores=NS)


def _g_compute_d4(si_vb, di_vb, d4_vb, x4t_v):
    for k in range(GC // 16):
        sl = pl.ds(k * 16, 16)
        si16 = si_vb[sl]
        di16 = di_vb[sl]
        for cc in range(3):
            a = plsc.load_gather(x4t_v, [si16 + (cc * N)])
            b2 = plsc.load_gather(x4t_v, [di16 + (cc * N)])
            d4_vb[cc, sl] = a - b2
        d4_vb[3, sl] = jnp.zeros((16,), jnp.float32)


@functools.partial(
    pl.kernel,
    out_type=[
        jax.ShapeDtypeStruct((E, H), jnp.float32),
        jax.ShapeDtypeStruct((E, H), jnp.float32),
        jax.ShapeDtypeStruct((4, E), jnp.float32),
    ],
    mesh=plsc.VectorSubcoreMesh(**_SC_MESH),
    scratch_types=[
        pltpu.VMEM((GC,), jnp.int32),
        pltpu.VMEM((GC,), jnp.int32),
        pltpu.VMEM((GC,), jnp.int32),
        pltpu.VMEM((GC,), jnp.int32),
        pltpu.VMEM((GC, H), jnp.float32),
        pltpu.VMEM((GC, H), jnp.float32),
        pltpu.VMEM((GC, H), jnp.float32),
        pltpu.VMEM((GC, H), jnp.float32),
        pltpu.VMEM((4, GC), jnp.float32),
        pltpu.VMEM((4, GC), jnp.float32),
        pltpu.VMEM((3 * N,), jnp.float32),
        pltpu.SemaphoreType.DMA,
        pltpu.SemaphoreType.DMA,
        pltpu.SemaphoreType.DMA,
        pltpu.SemaphoreType.DMA,
        pltpu.SemaphoreType.DMA,
        pltpu.SemaphoreType.DMA,
    ],
    compiler_params=pltpu.CompilerParams(needs_layout_passes=False),
)
def _sc_gather(p_hbm, q_hbm, x4t_hbm, si_hbm, di_hbm,
               g1_hbm, g2_hbm, d4_hbm,
               siA, siB, diA, diB, g1A, g1B, g2A, g2B, d4A, d4B, x4t_v,
               smi0, smi1, smg0, smg1, smw0, smw1):
    pltpu.sync_copy(x4t_hbm, x4t_v)
    wid = lax.axis_index("s") * NC + lax.axis_index("c")
    si_v = (siA, siB)
    di_v = (diA, diB)
    g1_v = (g1A, g1B)
    g2_v = (g2A, g2B)
    d4_v = (d4A, d4B)
    sem_i = (smi0, smi1)
    sem_g = (smg0, smg1)
    sem_w = (smw0, smw1)

    def idx_load(r, b):
        base = (r * NW + wid) * GC
        pltpu.async_copy(si_hbm.at[pl.ds(base, GC)], si_v[b], sem_i[b])
        pltpu.async_copy(di_hbm.at[pl.ds(base, GC)], di_v[b], sem_i[b])

    def idx_drain(b):
        pltpu.make_async_copy(si_hbm.at[pl.ds(0, GC)], si_v[b],
                              sem_i[b]).wait()
        pltpu.make_async_copy(di_hbm.at[pl.ds(0, GC)], di_v[b],
                              sem_i[b]).wait()

    def write_drain(b):
        pltpu.make_async_copy(g1_v[b], g1_hbm.at[pl.ds(0, GC)],
                              sem_w[b]).wait()
        pltpu.make_async_copy(g2_v[b], g2_hbm.at[pl.ds(0, GC)],
                              sem_w[b]).wait()
        pltpu.make_async_copy(d4_v[b], d4_hbm.at[:, pl.ds(0, GC)],
                              sem_w[b]).wait()

    idx_load(0, 0)

    def body_one(r, b):
        base = (r * NW + wid) * GC

        @pl.when(r >= 2)
        def _():
            write_drain(b)

        idx_drain(b)
        cp1 = pltpu.async_copy(p_hbm.at[si_v[b]], g1_v[b], sem_g[b])
        cp2 = pltpu.async_copy(q_hbm.at[di_v[b]], g2_v[b], sem_g[b])

        @pl.when(r + 1 < FULL_ROUNDS)
        def _():
            idx_load(r + 1, 1 - b)

        _g_compute_d4(si_v[b], di_v[b], d4_v[b], x4t_v)
        cp1.wait()
        cp2.wait()
        pltpu.async_copy(g1_v[b], g1_hbm.at[pl.ds(base, GC)], sem_w[b])
        pltpu.async_copy(g2_v[b], g2_hbm.at[pl.ds(base, GC)], sem_w[b])
        pltpu.async_copy(d4_v[b], d4_hbm.at[:, pl.ds(base, GC)],
                         sem_w[b])

    def body(i, _):
        body_one(2 * i, 0)
        body_one(2 * i + 1, 1)
        return 0

    lax.fori_loop(0, FULL_ROUNDS // 2, body, 0)
    write_drain(0)
    write_drain(1)

    @pl.when(wid < TAIL)
    def _():
        base = (FULL_ROUNDS * NW + wid) * GC
        pltpu.sync_copy(si_hbm.at[pl.ds(base, GC)], siA)
        pltpu.sync_copy(di_hbm.at[pl.ds(base, GC)], diA)
        cp1 = pltpu.async_copy(p_hbm.at[siA], g1A, smg0)
        cp2 = pltpu.async_copy(q_hbm.at[diA], g2A, smg0)
        _g_compute_d4(siA, diA, d4A, x4t_v)
        cp1.wait()
        cp2.wait()
        pltpu.sync_copy(g1A, g1_hbm.at[pl.ds(base, GC)])
        pltpu.sync_copy(g2A, g2_hbm.at[pl.ds(base, GC)])
        pltpu.sync_copy(d4A, d4_hbm.at[:, pl.ds(base, GC)])


def _silu(v):
    return v * jax.nn.sigmoid(v)


def _prep_body(h_ref, at_ref, bt_ref, p_ref, q_ref):
    hb = h_ref[...]
    p_ref[...] = jnp.dot(hb, at_ref[...], preferred_element_type=jnp.float32)
    q_ref[...] = jnp.dot(hb, bt_ref[...], preferred_element_type=jnp.float32)


def _node_prep(h, At, Bt):
    return pl.pallas_call(
        _prep_body,
        grid=(N // BN,),
        in_specs=[
            pl.BlockSpec((BN, H), lambda i: (i, 0)),
            pl.BlockSpec((H, H), lambda i: (0, 0)),
            pl.BlockSpec((H, H), lambda i: (0, 0)),
        ],
        out_specs=[
            pl.BlockSpec((BN, H), lambda i: (i, 0)),
            pl.BlockSpec((BN, H), lambda i: (i, 0)),
        ],
        out_shape=[
            jax.ShapeDtypeStruct((N, H), jnp.float32),
            jax.ShapeDtypeStruct((N, H), jnp.float32),
        ],
    )(h, At, Bt)


SC_CHUNKS = CHUNKS // NC
SC_ROUNDS = SC_CHUNKS // NS
SC_TAIL = SC_CHUNKS - SC_ROUNDS * NS
N2 = 10240
RT = N2 // NS
CR = N2 * 4 // H


@functools.partial(
    pl.kernel,
    out_type=jax.ShapeDtypeStruct((NC * N2, H), jnp.float32),
    mesh=plsc.VectorSubcoreMesh(**_SC_MESH),
    scratch_types=[
        pltpu.VMEM((1, GC), jnp.int32),
        pltpu.VMEM((1, GC), jnp.int32),
        pltpu.VMEM((GC, H), jnp.float32),
        pltpu.VMEM((GC, H), jnp.float32),
        pltpu.VMEM_SHARED((N2, H), jnp.float32),
        pltpu.SemaphoreType.DMA,
        pltpu.SemaphoreType.DMA,
    ],
    compiler_params=pltpu.CompilerParams(needs_layout_passes=False),
)
def _sc_scatter(m_hbm, di_hbm, aggp_hbm, diW0, diW1, m0, m1, acc_sh,
                sr0, sr1):
    c = lax.axis_index("c")
    s = lax.axis_index("s")
    di_v = (diW0, diW1)
    m_v = (m0, m1)
    sem_r = (sr0, sr1)

    def zrow(j, _):
        for k in range(H // 16):
            m0[j, pl.ds(k * 16, 16)] = jnp.zeros((16,), jnp.float32)
        return 0

    lax.fori_loop(0, GC, zrow, 0)

    rows0 = s * RT
    for j in range(RT // GC):
        pltpu.sync_copy(m0, acc_sh.at[pl.ds(rows0 + j * GC, GC)])
    plsc.subcore_barrier()

    n_chunks = SC_ROUNDS + 1

    def chunk_id(r):
        return c * SC_CHUNKS + r * NS + s

    def read(r, b):
        base = chunk_id(r) * GC
        pltpu.async_copy(di_hbm.at[pl.ds(base, GC)], di_v[b].at[0], sem_r[b])
        pltpu.async_copy(m_hbm.at[pl.ds(base, GC)], m_v[b], sem_r[b])

    def read_drain(b):
        pltpu.make_async_copy(di_hbm.at[pl.ds(0, GC)], di_v[b].at[0],
                              sem_r[b]).wait()
        pltpu.make_async_copy(m_hbm.at[pl.ds(0, GC)], m_v[b],
                              sem_r[b]).wait()

    def live(r):
        return jnp.logical_or(r < SC_ROUNDS,
                              jnp.logical_and(r == SC_ROUNDS, s < SC_TAIL))

    @pl.when(live(0))
    def _():
        read(0, 0)

    def body_one(r, b):
        @pl.when(live(r + 1))
        def _():
            read(r + 1, 1 - b)

        @pl.when(live(r))
        def _():
            read_drain(b)
            pltpu.sync_copy(m_v[b], acc_sh.at[di_v[b].at[0]], add=True)

    def body(i, _):
        body_one(2 * i, 0)
        body_one(2 * i + 1, 1)
        return 0

    lax.fori_loop(0, n_chunks // 2, body, 0)
    body_one(n_chunks - 1, 0)

    plsc.subcore_barrier()
    for j in range(RT // GC):
        r0 = rows0 + j * GC
        pltpu.sync_copy(acc_sh.at[pl.ds(r0, GC)], m0)
        pltpu.sync_copy(m0, aggp_hbm.at[pl.ds(c * N2 + r0, GC)])


def _cscatter_chunk(chunk, refs):
    (wd_hbm, di_hbm, di_v, wd_v, cacc_v) = refs
    base = chunk * GC
    pltpu.sync_copy(di_hbm.at[pl.ds(base, GC)], di_v)
    pltpu.sync_copy(wd_hbm.at[:, pl.ds(base, GC)], wd_v)
    for k in range(GC // 16):
        sl = pl.ds(k * 16, 16)
        dstv = di_v[sl]
        for cc in range(4):
            val = wd_v[cc, sl]
            plsc.addupdate_scatter(
                cacc_v, [jnp.full((16,), cc, jnp.int32), dstv], val)


@functools.partial(
    pl.kernel,
    out_type=jax.ShapeDtypeStruct((NW * 8, N2), jnp.float32),
    mesh=plsc.VectorSubcoreMesh(**_SC_MESH),
    scratch_types=[
        pltpu.VMEM((GC,), jnp.int32),
        pltpu.VMEM((4, GC), jnp.float32),
        pltpu.VMEM((8, N2), jnp.float32),
    ],
    compiler_params=pltpu.CompilerParams(needs_layout_passes=False),
)
def _sc_cscatter(wd_hbm, di_hbm, caccp_hbm, di_v, wd_v, cacc_v):
    c = lax.axis_index("c")
    s = lax.axis_index("s")
    wid = s * NC + c
    refs = (wd_hbm, di_hbm, di_v, wd_v, cacc_v)

    def zc(j, _):
        for cc in range(8):
            cacc_v[cc, pl.ds(j * 16, 16)] = jnp.zeros((16,), jnp.float32)
        return 0

    lax.fori_loop(0, N2 // 16, zc, 0)

    def body(r, _):
        _cscatter_chunk(r * NW + wid, refs)
        return 0

    lax.fori_loop(0, FULL_ROUNDS, body, 0)

    @pl.when(wid < TAIL)
    def _():
        _cscatter_chunk(FULL_ROUNDS * NW + wid, refs)

    pltpu.sync_copy(cacc_v, caccp_hbm.at[pl.ds(wid * 8, 8)])


def _edge_body(g1_ref, g2_ref, d4_ref, c2_ref, be1_ref, we2_ref,
               be2_ref, wc1_ref, bc1_ref, wc2_ref, m_ref, wd_ref):
    d4 = d4_ref[...]
    dsq = d4 * d4
    pre = (g1_ref[...] + g2_ref[...] + be1_ref[...]
           + lax.dot_general(dsq, c2_ref[...], (((0,), (0,)), ((), ())),
                             preferred_element_type=jnp.float32))
    m1 = _silu(pre)
    m = _silu(jnp.dot(m1, we2_ref[...], preferred_element_type=jnp.float32)
              + be2_ref[...])
    m_ref[...] = m
    c = _silu(jnp.dot(m, wc1_ref[...], preferred_element_type=jnp.float32)
              + bc1_ref[...])
    cw_w = lax.dot_general(wc2_ref[...], c, (((1,), (1,)), ((), ())),
                           preferred_element_type=jnp.float32)
    is3 = lax.broadcasted_iota(jnp.int32, d4.shape, 0) == 3
    wd_ref[...] = jnp.where(is3, 1.0, d4 * cw_w)


def _edge_mlp(g1, g2, d4, C2, b_e1, We2T, b_e2, Wc1T, b_c1, Wc2):
    full = lambda r, c: pl.BlockSpec((r, c), lambda i: (0, 0))
    return pl.pallas_call(
        _edge_body,
        grid=(E // BE,),
        in_specs=[
            pl.BlockSpec((BE, H), lambda i: (i, 0)),
            pl.BlockSpec((BE, H), lambda i: (i, 0)),
            pl.BlockSpec((4, BE), lambda i: (0, i)),
            full(4, H), full(1, H), full(H, H), full(1, H),
            full(H, H), full(1, H), full(1, H),
        ],
        out_specs=[
            pl.BlockSpec((BE, H), lambda i: (i, 0)),
            pl.BlockSpec((4, BE), lambda i: (0, i)),
        ],
        out_shape=[
            jax.ShapeDtypeStruct((E, H), jnp.float32),
            jax.ShapeDtypeStruct((4, E), jnp.float32),
        ],
    )(g1, g2, d4, C2, b_e1, We2T, b_e2, Wc1T, b_c1, Wc2)


def _node_body(h_ref, x4w_ref, aggp0_ref, aggp1_ref, cgp_ref,
               n1a_ref, n1b_ref, bn1_ref, n2_ref, bn2_ref, lnw_ref, lnb_ref,
               ho_ref, xo_ref):
    hb = h_ref[...]
    cg = jnp.sum(cgp_ref[...], axis=0)[:4, :]
    e41 = (lax.broadcasted_iota(jnp.int32, (4, 1), 0) == 3).astype(jnp.float32)
    cnt_col = jnp.maximum(
        lax.dot_general(cg, e41, (((0,), (0,)), ((), ())),
                        preferred_element_type=jnp.float32), 1.0)
    agg = (aggp0_ref[...] + aggp1_ref[...]) / cnt_col
    u = _silu(jnp.dot(hb, n1a_ref[...], preferred_element_type=jnp.float32)
              + jnp.dot(agg, n1b_ref[...], preferred_element_type=jnp.float32)
              + bn1_ref[...])
    t = jnp.dot(u, n2_ref[...], preferred_element_type=jnp.float32) + bn2_ref[...]
    ho = hb + t
    mu = jnp.mean(ho, axis=-1, keepdims=True)
    d = ho - mu
    var = jnp.mean(d * d, axis=-1, keepdims=True)
    ho_ref[...] = d * lax.rsqrt(var + 1e-5) * lnw_ref[...] + lnb_ref[...]
    cnt_w = jnp.maximum(cg[3:4, :], 1.0)
    xo_ref[...] = x4w_ref[...] + cg / cnt_w


def _node_update(h, x4w, aggp, cgp, N1aT, N1bT, b_n1, Wn2T, b_n2,
                 ln_w, ln_b):
    full = lambda r, c: pl.BlockSpec((r, c), lambda i: (0, 0))
    return pl.pallas_call(
        _node_body,
        grid=(N2 // BN2,),
        in_specs=[
            pl.BlockSpec((BN2, H), lambda i: (i, 0)),
            pl.BlockSpec((4, BN2), lambda i: (0, i)),
            pl.BlockSpec((BN2, H), lambda i: (i, 0)),
            pl.BlockSpec((BN2, H), lambda i: (i + N2 // BN2, 0)),
            pl.BlockSpec((NW, 8, BN2), lambda i: (0, 0, i)),
            full(H, H), full(H, H), full(1, H), full(H, H), full(1, H),
            full(1, H), full(1, H),
        ],
        out_specs=[
            pl.BlockSpec((BN2, H), lambda i: (i, 0)),
            pl.BlockSpec((4, BN2), lambda i: (0, i)),
        ],
        out_shape=[
            jax.ShapeDtypeStruct((N, H), jnp.float32),
            jax.ShapeDtypeStruct((4, N2), jnp.float32),
        ],
    )(h, x4w, aggp, aggp, cgp, N1aT, N1bT, b_n1, Wn2T, b_n2, ln_w, ln_b)


def kernel(h, x, edge_index, W_e1, b_e1, W_e2, b_e2, W_n1, b_n1, W_n2, b_n2,
           W_c1, b_c1, W_c2, ln_w, ln_b):
    src = edge_index[0]
    dst = edge_index[1]
    xt = x.T

    At = W_e1[:, :H].T
    Bt = W_e1[:, H:2 * H].T
    wr = W_e1[:, 2 * H].reshape(1, H)
    C2 = jnp.concatenate([jnp.broadcast_to(wr, (3, H)),
                          jnp.zeros((1, H), jnp.float32)], axis=0)

    p, q = _node_prep(h, At, Bt)

    g1, g2, d4 = _sc_gather(p, q, xt.reshape(3 * N), src, dst)

    m, wd = _edge_mlp(g1, g2, d4, C2, b_e1.reshape(1, H), W_e2.T,
                      b_e2.reshape(1, H), W_c1.T, b_c1.reshape(1, H), W_c2)

    aggp = _sc_scatter(m, dst)
    caccp = _sc_cscatter(wd, dst)
    cgp = caccp.reshape(NW, 8, N2)

    x4w = jnp.pad(xt, ((0, 1), (0, N2 - N)))
    ho, xo = _node_update(h, x4w, aggp, cgp,
                          W_n1[:, :H].T, W_n1[:, H:].T, b_n1.reshape(1, H),
                          W_n2.T, b_n2.reshape(1, H), ln_w.reshape(1, H),
                          ln_b.reshape(1, H))
    return (ho, xo[:3, :N].T)


# speedup vs baseline: 7.2122x; 1.0015x over previous
import functools

import jax
import jax.numpy as jnp
from jax import lax
from jax.experimental import pallas as pl
from jax.experimental.pallas import tpu as pltpu
from jax.experimental.pallas import tpu_sc as plsc

N = 10000
E = 320000
H = 128

BN = 400
BN2 = 2048
BE = 2560

NC = 2
NS = 16
NW = NC * NS
GC = 128
CHUNKS = E // GC
FULL_ROUNDS = CHUNKS // NW
TAIL = CHUNKS - FULL_ROUNDS * NW

_SC_MESH = dict(core_axis_name="c", subcore_axis_name="s",
                num_cores=NC, num_subcores=NS)


def _g_compute_d4(si_vb, di_vb, d4_vb, x4t_v):
    for k in range(GC // 16):
        sl = pl.ds(k * 16, 16)
        si16 = si_vb[sl]
        di16 = di_vb[sl]
        for cc in range(3):
            a = plsc.load_gather(x4t_v, [si16 + (cc * N)])
            b2 = plsc.load_gather(x4t_v, [di16 + (cc * N)])
            d4_vb[cc, sl] = a - b2
        d4_vb[3, sl] = jnp.zeros((16,), jnp.float32)


@functools.partial(
    pl.kernel,
    out_type=[
        jax.ShapeDtypeStruct((E, H), jnp.float32),
        jax.ShapeDtypeStruct((E, H), jnp.float32),
        jax.ShapeDtypeStruct((4, E), jnp.float32),
    ],
    mesh=plsc.VectorSubcoreMesh(**_SC_MESH),
    scratch_types=[
        pltpu.VMEM((GC,), jnp.int32),
        pltpu.VMEM((GC,), jnp.int32),
        pltpu.VMEM((GC,), jnp.int32),
        pltpu.VMEM((GC,), jnp.int32),
        pltpu.VMEM((GC, H), jnp.float32),
        pltpu.VMEM((GC, H), jnp.float32),
        pltpu.VMEM((GC, H), jnp.float32),
        pltpu.VMEM((GC, H), jnp.float32),
        pltpu.VMEM((4, GC), jnp.float32),
        pltpu.VMEM((4, GC), jnp.float32),
        pltpu.VMEM((3 * N,), jnp.float32),
        pltpu.SemaphoreType.DMA,
        pltpu.SemaphoreType.DMA,
        pltpu.SemaphoreType.DMA,
        pltpu.SemaphoreType.DMA,
        pltpu.SemaphoreType.DMA,
        pltpu.SemaphoreType.DMA,
    ],
    compiler_params=pltpu.CompilerParams(needs_layout_passes=False),
)
def _sc_gather(p_hbm, q_hbm, x4t_hbm, si_hbm, di_hbm,
               g1_hbm, g2_hbm, d4_hbm,
               siA, siB, diA, diB, g1A, g1B, g2A, g2B, d4A, d4B, x4t_v,
               smi0, smi1, smg0, smg1, smw0, smw1):
    pltpu.sync_copy(x4t_hbm, x4t_v)
    wid = lax.axis_index("s") * NC + lax.axis_index("c")
    si_v = (siA, siB)
    di_v = (diA, diB)
    g1_v = (g1A, g1B)
    g2_v = (g2A, g2B)
    d4_v = (d4A, d4B)
    sem_i = (smi0, smi1)
    sem_g = (smg0, smg1)
    sem_w = (smw0, smw1)

    def idx_load(r, b):
        base = (r * NW + wid) * GC
        pltpu.async_copy(si_hbm.at[pl.ds(base, GC)], si_v[b], sem_i[b])
        pltpu.async_copy(di_hbm.at[pl.ds(base, GC)], di_v[b], sem_i[b])

    def idx_drain(b):
        pltpu.make_async_copy(si_hbm.at[pl.ds(0, GC)], si_v[b],
                              sem_i[b]).wait()
        pltpu.make_async_copy(di_hbm.at[pl.ds(0, GC)], di_v[b],
                              sem_i[b]).wait()

    def write_drain(b):
        pltpu.make_async_copy(g1_v[b], g1_hbm.at[pl.ds(0, GC)],
                              sem_w[b]).wait()
        pltpu.make_async_copy(g2_v[b], g2_hbm.at[pl.ds(0, GC)],
                              sem_w[b]).wait()
        pltpu.make_async_copy(d4_v[b], d4_hbm.at[:, pl.ds(0, GC)],
                              sem_w[b]).wait()

    idx_load(0, 0)

    def body_one(r, b):
        base = (r * NW + wid) * GC

        @pl.when(r >= 2)
        def _():
            write_drain(b)

        idx_drain(b)
        cp1 = pltpu.async_copy(p_hbm.at[si_v[b]], g1_v[b], sem_g[b])
        cp2 = pltpu.async_copy(q_hbm.at[di_v[b]], g2_v[b], sem_g[b])

        @pl.when(r + 1 < FULL_ROUNDS)
        def _():
            idx_load(r + 1, 1 - b)

        _g_compute_d4(si_v[b], di_v[b], d4_v[b], x4t_v)
        cp1.wait()
        cp2.wait()
        pltpu.async_copy(g1_v[b], g1_hbm.at[pl.ds(base, GC)], sem_w[b])
        pltpu.async_copy(g2_v[b], g2_hbm.at[pl.ds(base, GC)], sem_w[b])
        pltpu.async_copy(d4_v[b], d4_hbm.at[:, pl.ds(base, GC)],
                         sem_w[b])

    def body(i, _):
        body_one(2 * i, 0)
        body_one(2 * i + 1, 1)
        return 0

    lax.fori_loop(0, FULL_ROUNDS // 2, body, 0)
    write_drain(0)
    write_drain(1)

    @pl.when(wid < TAIL)
    def _():
        base = (FULL_ROUNDS * NW + wid) * GC
        pltpu.sync_copy(si_hbm.at[pl.ds(base, GC)], siA)
        pltpu.sync_copy(di_hbm.at[pl.ds(base, GC)], diA)
        cp1 = pltpu.async_copy(p_hbm.at[siA], g1A, smg0)
        cp2 = pltpu.async_copy(q_hbm.at[diA], g2A, smg0)
        _g_compute_d4(siA, diA, d4A, x4t_v)
        cp1.wait()
        cp2.wait()
        pltpu.sync_copy(g1A, g1_hbm.at[pl.ds(base, GC)])
        pltpu.sync_copy(g2A, g2_hbm.at[pl.ds(base, GC)])
        pltpu.sync_copy(d4A, d4_hbm.at[:, pl.ds(base, GC)])


def _silu(v):
    return v * jax.nn.sigmoid(v)


def _prep_body(h_ref, at_ref, bt_ref, p_ref, q_ref):
    hb = h_ref[...]
    p_ref[...] = jnp.dot(hb, at_ref[...], preferred_element_type=jnp.float32)
    q_ref[...] = jnp.dot(hb, bt_ref[...], preferred_element_type=jnp.float32)


def _node_prep(h, At, Bt):
    return pl.pallas_call(
        _prep_body,
        grid=(N // BN,),
        in_specs=[
            pl.BlockSpec((BN, H), lambda i: (i, 0)),
            pl.BlockSpec((H, H), lambda i: (0, 0)),
            pl.BlockSpec((H, H), lambda i: (0, 0)),
        ],
        out_specs=[
            pl.BlockSpec((BN, H), lambda i: (i, 0)),
            pl.BlockSpec((BN, H), lambda i: (i, 0)),
        ],
        out_shape=[
            jax.ShapeDtypeStruct((N, H), jnp.float32),
            jax.ShapeDtypeStruct((N, H), jnp.float32),
        ],
    )(h, At, Bt)


SC_CHUNKS = CHUNKS // NC
SC_ROUNDS = SC_CHUNKS // NS
SC_TAIL = SC_CHUNKS - SC_ROUNDS * NS
N2 = 10240
RT = N2 // NS
CR = N2 * 4 // H


@functools.partial(
    pl.kernel,
    out_type=jax.ShapeDtypeStruct((NC * N2, H), jnp.float32),
    mesh=plsc.VectorSubcoreMesh(**_SC_MESH),
    scratch_types=[
        pltpu.VMEM((1, GC), jnp.int32),
        pltpu.VMEM((1, GC), jnp.int32),
        pltpu.VMEM((GC, H), jnp.float32),
        pltpu.VMEM((GC, H), jnp.float32),
        pltpu.VMEM_SHARED((N2, H), jnp.float32),
        pltpu.SemaphoreType.DMA,
        pltpu.SemaphoreType.DMA,
    ],
    compiler_params=pltpu.CompilerParams(needs_layout_passes=False),
)
def _sc_scatter(m_hbm, di_hbm, aggp_hbm, diW0, diW1, m0, m1, acc_sh,
                sr0, sr1):
    c = lax.axis_index("c")
    s = lax.axis_index("s")
    di_v = (diW0, diW1)
    m_v = (m0, m1)
    sem_r = (sr0, sr1)

    def zrow(j, _):
        for k in range(H // 16):
            m0[j, pl.ds(k * 16, 16)] = jnp.zeros((16,), jnp.float32)
        return 0

    lax.fori_loop(0, GC, zrow, 0)

    rows0 = s * RT
    for j in range(RT // GC):
        pltpu.sync_copy(m0, acc_sh.at[pl.ds(rows0 + j * GC, GC)])
    plsc.subcore_barrier()

    n_chunks = SC_ROUNDS + 1

    def chunk_id(r):
        return c * SC_CHUNKS + r * NS + s

    def read(r, b):
        base = chunk_id(r) * GC
        pltpu.async_copy(di_hbm.at[pl.ds(base, GC)], di_v[b].at[0], sem_r[b])
        pltpu.async_copy(m_hbm.at[pl.ds(base, GC)], m_v[b], sem_r[b])

    def read_drain(b):
        pltpu.make_async_copy(di_hbm.at[pl.ds(0, GC)], di_v[b].at[0],
                              sem_r[b]).wait()
        pltpu.make_async_copy(m_hbm.at[pl.ds(0, GC)], m_v[b],
                              sem_r[b]).wait()

    def live(r):
        return jnp.logical_or(r < SC_ROUNDS,
                              jnp.logical_and(r == SC_ROUNDS, s < SC_TAIL))

    @pl.when(live(0))
    def _():
        read(0, 0)

    def body_one(r, b):
        @pl.when(live(r + 1))
        def _():
            read(r + 1, 1 - b)

        @pl.when(live(r))
        def _():
            read_drain(b)
            pltpu.sync_copy(m_v[b], acc_sh.at[di_v[b].at[0]], add=True)

    def body(i, _):
        body_one(2 * i, 0)
        body_one(2 * i + 1, 1)
        return 0

    lax.fori_loop(0, n_chunks // 2, body, 0)
    body_one(n_chunks - 1, 0)

    plsc.subcore_barrier()
    for j in range(RT // GC):
        r0 = rows0 + j * GC
        pltpu.sync_copy(acc_sh.at[pl.ds(r0, GC)], m0)
        pltpu.sync_copy(m0, aggp_hbm.at[pl.ds(c * N2 + r0, GC)])


def _cscatter_chunk(chunk, refs):
    (wd_hbm, di_hbm, di_v, wd_v, cacc_v) = refs
    base = chunk * GC
    pltpu.sync_copy(di_hbm.at[pl.ds(base, GC)], di_v)
    pltpu.sync_copy(wd_hbm.at[:, pl.ds(base, GC)], wd_v)
    for k in range(GC // 16):
        sl = pl.ds(k * 16, 16)
        dstv = di_v[sl]
        for cc in range(4):
            val = wd_v[cc, sl]
            plsc.addupdate_scatter(
                cacc_v, [jnp.full((16,), cc, jnp.int32), dstv], val)


@functools.partial(
    pl.kernel,
    out_type=jax.ShapeDtypeStruct((NW * 8, N2), jnp.float32),
    mesh=plsc.VectorSubcoreMesh(**_SC_MESH),
    scratch_types=[
        pltpu.VMEM((GC,), jnp.int32),
        pltpu.VMEM((4, GC), jnp.float32),
        pltpu.VMEM((8, N2), jnp.float32),
    ],
    compiler_params=pltpu.CompilerParams(needs_layout_passes=False),
)
def _sc_cscatter(wd_hbm, di_hbm, caccp_hbm, di_v, wd_v, cacc_v):
    c = lax.axis_index("c")
    s = lax.axis_index("s")
    wid = s * NC + c
    refs = (wd_hbm, di_hbm, di_v, wd_v, cacc_v)

    def zc(j, _):
        for cc in range(8):
            cacc_v[cc, pl.ds(j * 16, 16)] = jnp.zeros((16,), jnp.float32)
        return 0

    lax.fori_loop(0, N2 // 16, zc, 0)

    def body(r, _):
        _cscatter_chunk(r * NW + wid, refs)
        return 0

    lax.fori_loop(0, FULL_ROUNDS, body, 0)

    @pl.when(wid < TAIL)
    def _():
        _cscatter_chunk(FULL_ROUNDS * NW + wid, refs)

    pltpu.sync_copy(cacc_v, caccp_hbm.at[pl.ds(wid * 8, 8)])


def _edge_body(g1_ref, g2_ref, d4_ref, c2_ref, be1_ref, we2_ref,
               be2_ref, wc1_ref, bc1_ref, wc2_ref, m_ref, wd_ref):
    bf = jnp.bfloat16
    d4 = d4_ref[...]
    dsq = (d4 * d4).astype(bf)
    pre = (g1_ref[...] + g2_ref[...] + be1_ref[...]
           + lax.dot_general(dsq, c2_ref[...].astype(bf),
                             (((0,), (0,)), ((), ())),
                             preferred_element_type=jnp.float32))
    m1 = _silu(pre).astype(bf)
    m = _silu(jnp.dot(m1, we2_ref[...].astype(bf),
                      preferred_element_type=jnp.float32) + be2_ref[...])
    m_ref[...] = m
    c = _silu(jnp.dot(m.astype(bf), wc1_ref[...].astype(bf),
                      preferred_element_type=jnp.float32) + bc1_ref[...])
    cw_w = lax.dot_general(wc2_ref[...].astype(bf), c.astype(bf),
                           (((1,), (1,)), ((), ())),
                           preferred_element_type=jnp.float32)
    is3 = lax.broadcasted_iota(jnp.int32, d4.shape, 0) == 3
    wd_ref[...] = jnp.where(is3, 1.0, d4 * cw_w)


def _edge_mlp(g1, g2, d4, C2, b_e1, We2T, b_e2, Wc1T, b_c1, Wc2):
    full = lambda r, c: pl.BlockSpec((r, c), lambda i: (0, 0))
    return pl.pallas_call(
        _edge_body,
        grid=(E // BE,),
        in_specs=[
            pl.BlockSpec((BE, H), lambda i: (i, 0)),
            pl.BlockSpec((BE, H), lambda i: (i, 0)),
            pl.BlockSpec((4, BE), lambda i: (0, i)),
            full(4, H), full(1, H), full(H, H), full(1, H),
            full(H, H), full(1, H), full(1, H),
        ],
        out_specs=[
            pl.BlockSpec((BE, H), lambda i: (i, 0)),
            pl.BlockSpec((4, BE), lambda i: (0, i)),
        ],
        out_shape=[
            jax.ShapeDtypeStruct((E, H), jnp.float32),
            jax.ShapeDtypeStruct((4, E), jnp.float32),
        ],
    )(g1, g2, d4, C2, b_e1, We2T, b_e2, Wc1T, b_c1, Wc2)


def _node_body(h_ref, x4w_ref, aggp0_ref, aggp1_ref, cgp_ref,
               n1a_ref, n1b_ref, bn1_ref, n2_ref, bn2_ref, lnw_ref, lnb_ref,
               ho_ref, xo_ref):
    hb = h_ref[...]
    cg = jnp.sum(cgp_ref[...], axis=0)[:4, :]
    e41 = (lax.broadcasted_iota(jnp.int32, (4, 1), 0) == 3).astype(jnp.float32)
    cnt_col = jnp.maximum(
        lax.dot_general(cg, e41, (((0,), (0,)), ((), ())),
                        preferred_element_type=jnp.float32), 1.0)
    agg = (aggp0_ref[...] + aggp1_ref[...]) / cnt_col
    u = _silu(jnp.dot(hb, n1a_ref[...], preferred_element_type=jnp.float32)
              + jnp.dot(agg, n1b_ref[...], preferred_element_type=jnp.float32)
              + bn1_ref[...])
    t = jnp.dot(u, n2_ref[...], preferred_element_type=jnp.float32) + bn2_ref[...]
    ho = hb + t
    mu = jnp.mean(ho, axis=-1, keepdims=True)
    d = ho - mu
    var = jnp.mean(d * d, axis=-1, keepdims=True)
    ho_ref[...] = d * lax.rsqrt(var + 1e-5) * lnw_ref[...] + lnb_ref[...]
    cnt_w = jnp.maximum(cg[3:4, :], 1.0)
    xo_ref[...] = x4w_ref[...] + cg / cnt_w


def _node_update(h, x4w, aggp, cgp, N1aT, N1bT, b_n1, Wn2T, b_n2,
                 ln_w, ln_b):
    full = lambda r, c: pl.BlockSpec((r, c), lambda i: (0, 0))
    return pl.pallas_call(
        _node_body,
        grid=(N2 // BN2,),
        in_specs=[
            pl.BlockSpec((BN2, H), lambda i: (i, 0)),
            pl.BlockSpec((4, BN2), lambda i: (0, i)),
            pl.BlockSpec((BN2, H), lambda i: (i, 0)),
            pl.BlockSpec((BN2, H), lambda i: (i + N2 // BN2, 0)),
            pl.BlockSpec((NW, 8, BN2), lambda i: (0, 0, i)),
            full(H, H), full(H, H), full(1, H), full(H, H), full(1, H),
            full(1, H), full(1, H),
        ],
        out_specs=[
            pl.BlockSpec((BN2, H), lambda i: (i, 0)),
            pl.BlockSpec((4, BN2), lambda i: (0, i)),
        ],
        out_shape=[
            jax.ShapeDtypeStruct((N, H), jnp.float32),
            jax.ShapeDtypeStruct((4, N2), jnp.float32),
        ],
    )(h, x4w, aggp, aggp, cgp, N1aT, N1bT, b_n1, Wn2T, b_n2, ln_w, ln_b)


def kernel(h, x, edge_index, W_e1, b_e1, W_e2, b_e2, W_n1, b_n1, W_n2, b_n2,
           W_c1, b_c1, W_c2, ln_w, ln_b):
    src = edge_index[0]
    dst = edge_index[1]
    xt = x.T

    At = W_e1[:, :H].T
    Bt = W_e1[:, H:2 * H].T
    wr = W_e1[:, 2 * H].reshape(1, H)
    C2 = jnp.concatenate([jnp.broadcast_to(wr, (3, H)),
                          jnp.zeros((1, H), jnp.float32)], axis=0)

    p, q = _node_prep(h, At, Bt)

    g1, g2, d4 = _sc_gather(p, q, xt.reshape(3 * N), src, dst)

    m, wd = _edge_mlp(g1, g2, d4, C2, b_e1.reshape(1, H), W_e2.T,
                      b_e2.reshape(1, H), W_c1.T, b_c1.reshape(1, H), W_c2)

    aggp = _sc_scatter(m, dst)
    caccp = _sc_cscatter(wd, dst)
    cgp = caccp.reshape(NW, 8, N2)

    x4w = jnp.pad(xt, ((0, 1), (0, N2 - N)))
    ho, xo = _node_update(h, x4w, aggp, cgp,
                          W_n1[:, :H].T, W_n1[:, H:].T, b_n1.reshape(1, H),
                          W_n2.T, b_n2.reshape(1, H), ln_w.reshape(1, H),
                          ln_b.reshape(1, H))
    return (ho, xo[:3, :N].T)


# speedup vs baseline: 7.7972x; 1.0811x over previous
import functools

import jax
import jax.numpy as jnp
from jax import lax
from jax.experimental import pallas as pl
from jax.experimental.pallas import tpu as pltpu
from jax.experimental.pallas import tpu_sc as plsc

N = 10000
E = 320000
H = 128

BN = 400
BN2 = 2048
BE = 6400

NC = 2
NS = 16
NW = NC * NS
GC = 128
CHUNKS = E // GC
FULL_ROUNDS = CHUNKS // NW
TAIL = CHUNKS - FULL_ROUNDS * NW

_SC_MESH = dict(core_axis_name="c", subcore_axis_name="s",
                num_cores=NC, num_subcores=NS)


def _g_compute_d4(si_vb, di_vb, d4_vb, x4t_v):
    for k in range(GC // 16):
        sl = pl.ds(k * 16, 16)
        si16 = si_vb[sl]
        di16 = di_vb[sl]
        for cc in range(3):
            a = plsc.load_gather(x4t_v, [si16 + (cc * N)])
            b2 = plsc.load_gather(x4t_v, [di16 + (cc * N)])
            d4_vb[cc, sl] = a - b2
        d4_vb[3, sl] = jnp.zeros((16,), jnp.float32)


@functools.partial(
    pl.kernel,
    out_type=[
        jax.ShapeDtypeStruct((E, H), jnp.float32),
        jax.ShapeDtypeStruct((E, H), jnp.float32),
        jax.ShapeDtypeStruct((4, E), jnp.float32),
    ],
    mesh=plsc.VectorSubcoreMesh(**_SC_MESH),
    scratch_types=[
        pltpu.VMEM((GC,), jnp.int32),
        pltpu.VMEM((GC,), jnp.int32),
        pltpu.VMEM((GC,), jnp.int32),
        pltpu.VMEM((GC,), jnp.int32),
        pltpu.VMEM((GC, H), jnp.float32),
        pltpu.VMEM((GC, H), jnp.float32),
        pltpu.VMEM((GC, H), jnp.float32),
        pltpu.VMEM((GC, H), jnp.float32),
        pltpu.VMEM((4, GC), jnp.float32),
        pltpu.VMEM((4, GC), jnp.float32),
        pltpu.VMEM((3 * N,), jnp.float32),
        pltpu.SemaphoreType.DMA,
        pltpu.SemaphoreType.DMA,
        pltpu.SemaphoreType.DMA,
        pltpu.SemaphoreType.DMA,
        pltpu.SemaphoreType.DMA,
        pltpu.SemaphoreType.DMA,
    ],
    compiler_params=pltpu.CompilerParams(needs_layout_passes=False),
)
def _sc_gather(p_hbm, q_hbm, x4t_hbm, si_hbm, di_hbm,
               g1_hbm, g2_hbm, d4_hbm,
               siA, siB, diA, diB, g1A, g1B, g2A, g2B, d4A, d4B, x4t_v,
               smi0, smi1, smg0, smg1, smw0, smw1):
    pltpu.sync_copy(x4t_hbm, x4t_v)
    wid = lax.axis_index("s") * NC + lax.axis_index("c")
    si_v = (siA, siB)
    di_v = (diA, diB)
    g1_v = (g1A, g1B)
    g2_v = (g2A, g2B)
    d4_v = (d4A, d4B)
    sem_i = (smi0, smi1)
    sem_g = (smg0, smg1)
    sem_w = (smw0, smw1)

    def idx_load(r, b):
        base = (r * NW + wid) * GC
        pltpu.async_copy(si_hbm.at[pl.ds(base, GC)], si_v[b], sem_i[b])
        pltpu.async_copy(di_hbm.at[pl.ds(base, GC)], di_v[b], sem_i[b])

    def idx_drain(b):
        pltpu.make_async_copy(si_hbm.at[pl.ds(0, GC)], si_v[b],
                              sem_i[b]).wait()
        pltpu.make_async_copy(di_hbm.at[pl.ds(0, GC)], di_v[b],
                              sem_i[b]).wait()

    def write_drain(b):
        pltpu.make_async_copy(g1_v[b], g1_hbm.at[pl.ds(0, GC)],
                              sem_w[b]).wait()
        pltpu.make_async_copy(g2_v[b], g2_hbm.at[pl.ds(0, GC)],
                              sem_w[b]).wait()
        pltpu.make_async_copy(d4_v[b], d4_hbm.at[:, pl.ds(0, GC)],
                              sem_w[b]).wait()

    idx_load(0, 0)

    def body_one(r, b):
        base = (r * NW + wid) * GC

        @pl.when(r >= 2)
        def _():
            write_drain(b)

        idx_drain(b)
        cp1 = pltpu.async_copy(p_hbm.at[si_v[b]], g1_v[b], sem_g[b])
        cp2 = pltpu.async_copy(q_hbm.at[di_v[b]], g2_v[b], sem_g[b])

        @pl.when(r + 1 < FULL_ROUNDS)
        def _():
            idx_load(r + 1, 1 - b)

        _g_compute_d4(si_v[b], di_v[b], d4_v[b], x4t_v)
        cp1.wait()
        cp2.wait()
        pltpu.async_copy(g1_v[b], g1_hbm.at[pl.ds(base, GC)], sem_w[b])
        pltpu.async_copy(g2_v[b], g2_hbm.at[pl.ds(base, GC)], sem_w[b])
        pltpu.async_copy(d4_v[b], d4_hbm.at[:, pl.ds(base, GC)],
                         sem_w[b])

    def body(i, _):
        body_one(2 * i, 0)
        body_one(2 * i + 1, 1)
        return 0

    lax.fori_loop(0, FULL_ROUNDS // 2, body, 0)
    write_drain(0)
    write_drain(1)

    @pl.when(wid < TAIL)
    def _():
        base = (FULL_ROUNDS * NW + wid) * GC
        pltpu.sync_copy(si_hbm.at[pl.ds(base, GC)], siA)
        pltpu.sync_copy(di_hbm.at[pl.ds(base, GC)], diA)
        cp1 = pltpu.async_copy(p_hbm.at[siA], g1A, smg0)
        cp2 = pltpu.async_copy(q_hbm.at[diA], g2A, smg0)
        _g_compute_d4(siA, diA, d4A, x4t_v)
        cp1.wait()
        cp2.wait()
        pltpu.sync_copy(g1A, g1_hbm.at[pl.ds(base, GC)])
        pltpu.sync_copy(g2A, g2_hbm.at[pl.ds(base, GC)])
        pltpu.sync_copy(d4A, d4_hbm.at[:, pl.ds(base, GC)])


def _silu(v):
    return v * jax.nn.sigmoid(v)


def _prep_body(h_ref, at_ref, bt_ref, p_ref, q_ref):
    hb = h_ref[...]
    p_ref[...] = jnp.dot(hb, at_ref[...], preferred_element_type=jnp.float32)
    q_ref[...] = jnp.dot(hb, bt_ref[...], preferred_element_type=jnp.float32)


def _node_prep(h, At, Bt):
    return pl.pallas_call(
        _prep_body,
        grid=(N // BN,),
        in_specs=[
            pl.BlockSpec((BN, H), lambda i: (i, 0)),
            pl.BlockSpec((H, H), lambda i: (0, 0)),
            pl.BlockSpec((H, H), lambda i: (0, 0)),
        ],
        out_specs=[
            pl.BlockSpec((BN, H), lambda i: (i, 0)),
            pl.BlockSpec((BN, H), lambda i: (i, 0)),
        ],
        out_shape=[
            jax.ShapeDtypeStruct((N, H), jnp.float32),
            jax.ShapeDtypeStruct((N, H), jnp.float32),
        ],
    )(h, At, Bt)


SC_CHUNKS = CHUNKS // NC
SC_ROUNDS = SC_CHUNKS // NS
SC_TAIL = SC_CHUNKS - SC_ROUNDS * NS
N2 = 10240
RT = N2 // NS
CR = N2 * 4 // H


@functools.partial(
    pl.kernel,
    out_type=jax.ShapeDtypeStruct((NC * N2, H), jnp.float32),
    mesh=plsc.VectorSubcoreMesh(**_SC_MESH),
    scratch_types=[
        pltpu.VMEM((1, GC), jnp.int32),
        pltpu.VMEM((1, GC), jnp.int32),
        pltpu.VMEM((GC, H), jnp.float32),
        pltpu.VMEM((GC, H), jnp.float32),
        pltpu.VMEM_SHARED((N2, H), jnp.float32),
        pltpu.SemaphoreType.DMA,
        pltpu.SemaphoreType.DMA,
    ],
    compiler_params=pltpu.CompilerParams(needs_layout_passes=False),
)
def _sc_scatter(m_hbm, di_hbm, aggp_hbm, diW0, diW1, m0, m1, acc_sh,
                sr0, sr1):
    c = lax.axis_index("c")
    s = lax.axis_index("s")
    di_v = (diW0, diW1)
    m_v = (m0, m1)
    sem_r = (sr0, sr1)

    def zrow(j, _):
        for k in range(H // 16):
            m0[j, pl.ds(k * 16, 16)] = jnp.zeros((16,), jnp.float32)
        return 0

    lax.fori_loop(0, GC, zrow, 0)

    rows0 = s * RT
    for j in range(RT // GC):
        pltpu.sync_copy(m0, acc_sh.at[pl.ds(rows0 + j * GC, GC)])
    plsc.subcore_barrier()

    n_chunks = SC_ROUNDS + 1

    def chunk_id(r):
        return c * SC_CHUNKS + r * NS + s

    def read(r, b):
        base = chunk_id(r) * GC
        pltpu.async_copy(di_hbm.at[pl.ds(base, GC)], di_v[b].at[0], sem_r[b])
        pltpu.async_copy(m_hbm.at[pl.ds(base, GC)], m_v[b], sem_r[b])

    def read_drain(b):
        pltpu.make_async_copy(di_hbm.at[pl.ds(0, GC)], di_v[b].at[0],
                              sem_r[b]).wait()
        pltpu.make_async_copy(m_hbm.at[pl.ds(0, GC)], m_v[b],
                              sem_r[b]).wait()

    def live(r):
        return jnp.logical_or(r < SC_ROUNDS,
                              jnp.logical_and(r == SC_ROUNDS, s < SC_TAIL))

    @pl.when(live(0))
    def _():
        read(0, 0)

    def body_one(r, b):
        @pl.when(live(r + 1))
        def _():
            read(r + 1, 1 - b)

        @pl.when(live(r))
        def _():
            read_drain(b)
            pltpu.sync_copy(m_v[b], acc_sh.at[di_v[b].at[0]], add=True)

    def body(i, _):
        body_one(2 * i, 0)
        body_one(2 * i + 1, 1)
        return 0

    lax.fori_loop(0, n_chunks // 2, body, 0)
    body_one(n_chunks - 1, 0)

    plsc.subcore_barrier()
    for j in range(RT // GC):
        r0 = rows0 + j * GC
        pltpu.sync_copy(acc_sh.at[pl.ds(r0, GC)], m0)
        pltpu.sync_copy(m0, aggp_hbm.at[pl.ds(c * N2 + r0, GC)])


def _cscatter_chunk(chunk, refs):
    (wd_hbm, di_hbm, di_v, wd_v, cacc_v) = refs
    base = chunk * GC
    pltpu.sync_copy(di_hbm.at[pl.ds(base, GC)], di_v)
    pltpu.sync_copy(wd_hbm.at[:, pl.ds(base, GC)], wd_v)
    for k in range(GC // 16):
        sl = pl.ds(k * 16, 16)
        dstv = di_v[sl]
        for cc in range(4):
            val = wd_v[cc, sl]
            plsc.addupdate_scatter(
                cacc_v, [jnp.full((16,), cc, jnp.int32), dstv], val)


@functools.partial(
    pl.kernel,
    out_type=jax.ShapeDtypeStruct((NW * 8, N2), jnp.float32),
    mesh=plsc.VectorSubcoreMesh(**_SC_MESH),
    scratch_types=[
        pltpu.VMEM((GC,), jnp.int32),
        pltpu.VMEM((4, GC), jnp.float32),
        pltpu.VMEM((8, N2), jnp.float32),
    ],
    compiler_params=pltpu.CompilerParams(needs_layout_passes=False),
)
def _sc_cscatter(wd_hbm, di_hbm, caccp_hbm, di_v, wd_v, cacc_v):
    c = lax.axis_index("c")
    s = lax.axis_index("s")
    wid = s * NC + c
    refs = (wd_hbm, di_hbm, di_v, wd_v, cacc_v)

    def zc(j, _):
        for cc in range(8):
            cacc_v[cc, pl.ds(j * 16, 16)] = jnp.zeros((16,), jnp.float32)
        return 0

    lax.fori_loop(0, N2 // 16, zc, 0)

    def body(r, _):
        _cscatter_chunk(r * NW + wid, refs)
        return 0

    lax.fori_loop(0, FULL_ROUNDS, body, 0)

    @pl.when(wid < TAIL)
    def _():
        _cscatter_chunk(FULL_ROUNDS * NW + wid, refs)

    pltpu.sync_copy(cacc_v, caccp_hbm.at[pl.ds(wid * 8, 8)])


def _edge_body(g1_ref, g2_ref, d4_ref, c2_ref, be1_ref, we2_ref,
               be2_ref, wc1_ref, bc1_ref, wc2_ref, m_ref, wd_ref):
    bf = jnp.bfloat16
    d4 = d4_ref[...]
    dsq = (d4 * d4).astype(bf)
    pre = (g1_ref[...] + g2_ref[...] + be1_ref[...]
           + lax.dot_general(dsq, c2_ref[...].astype(bf),
                             (((0,), (0,)), ((), ())),
                             preferred_element_type=jnp.float32))
    m1 = _silu(pre).astype(bf)
    m = _silu(jnp.dot(m1, we2_ref[...].astype(bf),
                      preferred_element_type=jnp.float32) + be2_ref[...])
    m_ref[...] = m
    c = _silu(jnp.dot(m.astype(bf), wc1_ref[...].astype(bf),
                      preferred_element_type=jnp.float32) + bc1_ref[...])
    cw_w = lax.dot_general(wc2_ref[...].astype(bf), c.astype(bf),
                           (((1,), (1,)), ((), ())),
                           preferred_element_type=jnp.float32)
    is3 = lax.broadcasted_iota(jnp.int32, d4.shape, 0) == 3
    wd_ref[...] = jnp.where(is3, 1.0, d4 * cw_w)


def _edge_mlp(g1, g2, d4, C2, b_e1, We2T, b_e2, Wc1T, b_c1, Wc2):
    full = lambda r, c: pl.BlockSpec((r, c), lambda i: (0, 0))
    return pl.pallas_call(
        _edge_body,
        grid=(E // BE,),
        in_specs=[
            pl.BlockSpec((BE, H), lambda i: (i, 0)),
            pl.BlockSpec((BE, H), lambda i: (i, 0)),
            pl.BlockSpec((4, BE), lambda i: (0, i)),
            full(4, H), full(1, H), full(H, H), full(1, H),
            full(H, H), full(1, H), full(1, H),
        ],
        out_specs=[
            pl.BlockSpec((BE, H), lambda i: (i, 0)),
            pl.BlockSpec((4, BE), lambda i: (0, i)),
        ],
        out_shape=[
            jax.ShapeDtypeStruct((E, H), jnp.float32),
            jax.ShapeDtypeStruct((4, E), jnp.float32),
        ],
    )(g1, g2, d4, C2, b_e1, We2T, b_e2, Wc1T, b_c1, Wc2)


def _node_body(h_ref, x4w_ref, aggp0_ref, aggp1_ref, cgp_ref,
               n1a_ref, n1b_ref, bn1_ref, n2_ref, bn2_ref, lnw_ref, lnb_ref,
               ho_ref, xo_ref):
    hb = h_ref[...]
    cg = jnp.sum(cgp_ref[...], axis=0)[:4, :]
    e41 = (lax.broadcasted_iota(jnp.int32, (4, 1), 0) == 3).astype(jnp.float32)
    cnt_col = jnp.maximum(
        lax.dot_general(cg, e41, (((0,), (0,)), ((), ())),
                        preferred_element_type=jnp.float32), 1.0)
    agg = (aggp0_ref[...] + aggp1_ref[...]) / cnt_col
    u = _silu(jnp.dot(hb, n1a_ref[...], preferred_element_type=jnp.float32)
              + jnp.dot(agg, n1b_ref[...], preferred_element_type=jnp.float32)
              + bn1_ref[...])
    t = jnp.dot(u, n2_ref[...], preferred_element_type=jnp.float32) + bn2_ref[...]
    ho = hb + t
    mu = jnp.mean(ho, axis=-1, keepdims=True)
    d = ho - mu
    var = jnp.mean(d * d, axis=-1, keepdims=True)
    ho_ref[...] = d * lax.rsqrt(var + 1e-5) * lnw_ref[...] + lnb_ref[...]
    cnt_w = jnp.maximum(cg[3:4, :], 1.0)
    xo_ref[...] = x4w_ref[...] + cg / cnt_w


def _node_update(h, x4w, aggp, cgp, N1aT, N1bT, b_n1, Wn2T, b_n2,
                 ln_w, ln_b):
    full = lambda r, c: pl.BlockSpec((r, c), lambda i: (0, 0))
    return pl.pallas_call(
        _node_body,
        grid=(N2 // BN2,),
        in_specs=[
            pl.BlockSpec((BN2, H), lambda i: (i, 0)),
            pl.BlockSpec((4, BN2), lambda i: (0, i)),
            pl.BlockSpec((BN2, H), lambda i: (i, 0)),
            pl.BlockSpec((BN2, H), lambda i: (i + N2 // BN2, 0)),
            pl.BlockSpec((NW, 8, BN2), lambda i: (0, 0, i)),
            full(H, H), full(H, H), full(1, H), full(H, H), full(1, H),
            full(1, H), full(1, H),
        ],
        out_specs=[
            pl.BlockSpec((BN2, H), lambda i: (i, 0)),
            pl.BlockSpec((4, BN2), lambda i: (0, i)),
        ],
        out_shape=[
            jax.ShapeDtypeStruct((N, H), jnp.float32),
            jax.ShapeDtypeStruct((4, N2), jnp.float32),
        ],
    )(h, x4w, aggp, aggp, cgp, N1aT, N1bT, b_n1, Wn2T, b_n2, ln_w, ln_b)


def kernel(h, x, edge_index, W_e1, b_e1, W_e2, b_e2, W_n1, b_n1, W_n2, b_n2,
           W_c1, b_c1, W_c2, ln_w, ln_b):
    src = edge_index[0]
    dst = edge_index[1]
    xt = x.T

    At = W_e1[:, :H].T
    Bt = W_e1[:, H:2 * H].T
    wr = W_e1[:, 2 * H].reshape(1, H)
    C2 = jnp.concatenate([jnp.broadcast_to(wr, (3, H)),
                          jnp.zeros((1, H), jnp.float32)], axis=0)

    p, q = _node_prep(h, At, Bt)

    g1, g2, d4 = _sc_gather(p, q, xt.reshape(3 * N), src, dst)

    m, wd = _edge_mlp(g1, g2, d4, C2, b_e1.reshape(1, H), W_e2.T,
                      b_e2.reshape(1, H), W_c1.T, b_c1.reshape(1, H), W_c2)

    aggp = _sc_scatter(m, dst)
    caccp = _sc_cscatter(wd, dst)
    cgp = caccp.reshape(NW, 8, N2)

    x4w = jnp.pad(xt, ((0, 1), (0, N2 - N)))
    ho, xo = _node_update(h, x4w, aggp, cgp,
                          W_n1[:, :H].T, W_n1[:, H:].T, b_n1.reshape(1, H),
                          W_n2.T, b_n2.reshape(1, H), ln_w.reshape(1, H),
                          ln_b.reshape(1, H))
    return (ho, xo[:3, :N].T)


# speedup vs baseline: 7.9695x; 1.0221x over previous
import functools

import jax
import jax.numpy as jnp
from jax import lax
from jax.experimental import pallas as pl
from jax.experimental.pallas import tpu as pltpu
from jax.experimental.pallas import tpu_sc as plsc

N = 10000
E = 320000
H = 128

BN = 400
BN2 = 2048
BE = 12800

NC = 2
NS = 16
NW = NC * NS
GC = 128
CHUNKS = E // GC
FULL_ROUNDS = CHUNKS // NW
TAIL = CHUNKS - FULL_ROUNDS * NW

_SC_MESH = dict(core_axis_name="c", subcore_axis_name="s",
                num_cores=NC, num_subcores=NS)


def _g_compute_d4(si_vb, di_vb, d4_vb, x4t_v):
    for k in range(GC // 16):
        sl = pl.ds(k * 16, 16)
        si16 = si_vb[sl]
        di16 = di_vb[sl]
        for cc in range(3):
            a = plsc.load_gather(x4t_v, [si16 + (cc * N)])
            b2 = plsc.load_gather(x4t_v, [di16 + (cc * N)])
            d4_vb[cc, sl] = a - b2
        d4_vb[3, sl] = jnp.zeros((16,), jnp.float32)


@functools.partial(
    pl.kernel,
    out_type=[
        jax.ShapeDtypeStruct((E, H), jnp.float32),
        jax.ShapeDtypeStruct((E, H), jnp.float32),
        jax.ShapeDtypeStruct((4, E), jnp.float32),
    ],
    mesh=plsc.VectorSubcoreMesh(**_SC_MESH),
    scratch_types=[
        pltpu.VMEM((GC,), jnp.int32),
        pltpu.VMEM((GC,), jnp.int32),
        pltpu.VMEM((GC,), jnp.int32),
        pltpu.VMEM((GC,), jnp.int32),
        pltpu.VMEM((GC, H), jnp.float32),
        pltpu.VMEM((GC, H), jnp.float32),
        pltpu.VMEM((GC, H), jnp.float32),
        pltpu.VMEM((GC, H), jnp.float32),
        pltpu.VMEM((4, GC), jnp.float32),
        pltpu.VMEM((4, GC), jnp.float32),
        pltpu.VMEM((3 * N,), jnp.float32),
        pltpu.SemaphoreType.DMA,
        pltpu.SemaphoreType.DMA,
        pltpu.SemaphoreType.DMA,
        pltpu.SemaphoreType.DMA,
        pltpu.SemaphoreType.DMA,
        pltpu.SemaphoreType.DMA,
    ],
    compiler_params=pltpu.CompilerParams(needs_layout_passes=False),
)
def _sc_gather(p_hbm, q_hbm, x4t_hbm, si_hbm, di_hbm,
               g1_hbm, g2_hbm, d4_hbm,
               siA, siB, diA, diB, g1A, g1B, g2A, g2B, d4A, d4B, x4t_v,
               smi0, smi1, smg0, smg1, smw0, smw1):
    pltpu.sync_copy(x4t_hbm, x4t_v)
    wid = lax.axis_index("s") * NC + lax.axis_index("c")
    si_v = (siA, siB)
    di_v = (diA, diB)
    g1_v = (g1A, g1B)
    g2_v = (g2A, g2B)
    d4_v = (d4A, d4B)
    sem_i = (smi0, smi1)
    sem_g = (smg0, smg1)
    sem_w = (smw0, smw1)

    def idx_load(r, b):
        base = (r * NW + wid) * GC
        pltpu.async_copy(si_hbm.at[pl.ds(base, GC)], si_v[b], sem_i[b])
        pltpu.async_copy(di_hbm.at[pl.ds(base, GC)], di_v[b], sem_i[b])

    def idx_drain(b):
        pltpu.make_async_copy(si_hbm.at[pl.ds(0, GC)], si_v[b],
                              sem_i[b]).wait()
        pltpu.make_async_copy(di_hbm.at[pl.ds(0, GC)], di_v[b],
                              sem_i[b]).wait()

    def write_drain(b):
        pltpu.make_async_copy(g1_v[b], g1_hbm.at[pl.ds(0, GC)],
                              sem_w[b]).wait()
        pltpu.make_async_copy(g2_v[b], g2_hbm.at[pl.ds(0, GC)],
                              sem_w[b]).wait()
        pltpu.make_async_copy(d4_v[b], d4_hbm.at[:, pl.ds(0, GC)],
                              sem_w[b]).wait()

    idx_load(0, 0)

    def body_one(r, b):
        base = (r * NW + wid) * GC

        @pl.when(r >= 2)
        def _():
            write_drain(b)

        idx_drain(b)
        cp1 = pltpu.async_copy(p_hbm.at[si_v[b]], g1_v[b], sem_g[b])
        cp2 = pltpu.async_copy(q_hbm.at[di_v[b]], g2_v[b], sem_g[b])

        @pl.when(r + 1 < FULL_ROUNDS)
        def _():
            idx_load(r + 1, 1 - b)

        _g_compute_d4(si_v[b], di_v[b], d4_v[b], x4t_v)
        cp1.wait()
        cp2.wait()
        pltpu.async_copy(g1_v[b], g1_hbm.at[pl.ds(base, GC)], sem_w[b])
        pltpu.async_copy(g2_v[b], g2_hbm.at[pl.ds(base, GC)], sem_w[b])
        pltpu.async_copy(d4_v[b], d4_hbm.at[:, pl.ds(base, GC)],
                         sem_w[b])

    def body(i, _):
        body_one(2 * i, 0)
        body_one(2 * i + 1, 1)
        return 0

    lax.fori_loop(0, FULL_ROUNDS // 2, body, 0)
    write_drain(0)
    write_drain(1)

    @pl.when(wid < TAIL)
    def _():
        base = (FULL_ROUNDS * NW + wid) * GC
        pltpu.sync_copy(si_hbm.at[pl.ds(base, GC)], siA)
        pltpu.sync_copy(di_hbm.at[pl.ds(base, GC)], diA)
        cp1 = pltpu.async_copy(p_hbm.at[siA], g1A, smg0)
        cp2 = pltpu.async_copy(q_hbm.at[diA], g2A, smg0)
        _g_compute_d4(siA, diA, d4A, x4t_v)
        cp1.wait()
        cp2.wait()
        pltpu.sync_copy(g1A, g1_hbm.at[pl.ds(base, GC)])
        pltpu.sync_copy(g2A, g2_hbm.at[pl.ds(base, GC)])
        pltpu.sync_copy(d4A, d4_hbm.at[:, pl.ds(base, GC)])


def _silu(v):
    return v * jax.nn.sigmoid(v)


def _prep_body(h_ref, at_ref, bt_ref, p_ref, q_ref):
    hb = h_ref[...]
    p_ref[...] = jnp.dot(hb, at_ref[...], preferred_element_type=jnp.float32)
    q_ref[...] = jnp.dot(hb, bt_ref[...], preferred_element_type=jnp.float32)


def _node_prep(h, At, Bt):
    return pl.pallas_call(
        _prep_body,
        grid=(N // BN,),
        in_specs=[
            pl.BlockSpec((BN, H), lambda i: (i, 0)),
            pl.BlockSpec((H, H), lambda i: (0, 0)),
            pl.BlockSpec((H, H), lambda i: (0, 0)),
        ],
        out_specs=[
            pl.BlockSpec((BN, H), lambda i: (i, 0)),
            pl.BlockSpec((BN, H), lambda i: (i, 0)),
        ],
        out_shape=[
            jax.ShapeDtypeStruct((N, H), jnp.float32),
            jax.ShapeDtypeStruct((N, H), jnp.float32),
        ],
    )(h, At, Bt)


SC_CHUNKS = CHUNKS // NC
SC_ROUNDS = SC_CHUNKS // NS
SC_TAIL = SC_CHUNKS - SC_ROUNDS * NS
N2 = 10240
RT = N2 // NS
CR = N2 * 4 // H


@functools.partial(
    pl.kernel,
    out_type=jax.ShapeDtypeStruct((NC * N2, H), jnp.float32),
    mesh=plsc.VectorSubcoreMesh(**_SC_MESH),
    scratch_types=[
        pltpu.VMEM((1, GC), jnp.int32),
        pltpu.VMEM((1, GC), jnp.int32),
        pltpu.VMEM((GC, H), jnp.float32),
        pltpu.VMEM((GC, H), jnp.float32),
        pltpu.VMEM_SHARED((N2, H), jnp.float32),
        pltpu.SemaphoreType.DMA,
        pltpu.SemaphoreType.DMA,
    ],
    compiler_params=pltpu.CompilerParams(needs_layout_passes=False),
)
def _sc_scatter(m_hbm, di_hbm, aggp_hbm, diW0, diW1, m0, m1, acc_sh,
                sr0, sr1):
    c = lax.axis_index("c")
    s = lax.axis_index("s")
    di_v = (diW0, diW1)
    m_v = (m0, m1)
    sem_r = (sr0, sr1)

    def zrow(j, _):
        for k in range(H // 16):
            m0[j, pl.ds(k * 16, 16)] = jnp.zeros((16,), jnp.float32)
        return 0

    lax.fori_loop(0, GC, zrow, 0)

    rows0 = s * RT
    for j in range(RT // GC):
        pltpu.sync_copy(m0, acc_sh.at[pl.ds(rows0 + j * GC, GC)])
    plsc.subcore_barrier()

    n_chunks = SC_ROUNDS + 1

    def chunk_id(r):
        return c * SC_CHUNKS + r * NS + s

    def read(r, b):
        base = chunk_id(r) * GC
        pltpu.async_copy(di_hbm.at[pl.ds(base, GC)], di_v[b].at[0], sem_r[b])
        pltpu.async_copy(m_hbm.at[pl.ds(base, GC)], m_v[b], sem_r[b])

    def read_drain(b):
        pltpu.make_async_copy(di_hbm.at[pl.ds(0, GC)], di_v[b].at[0],
                              sem_r[b]).wait()
        pltpu.make_async_copy(m_hbm.at[pl.ds(0, GC)], m_v[b],
                              sem_r[b]).wait()

    def live(r):
        return jnp.logical_or(r < SC_ROUNDS,
                              jnp.logical_and(r == SC_ROUNDS, s < SC_TAIL))

    @pl.when(live(0))
    def _():
        read(0, 0)

    def body_one(r, b):
        @pl.when(live(r + 1))
        def _():
            read(r + 1, 1 - b)

        @pl.when(live(r))
        def _():
            read_drain(b)
            pltpu.sync_copy(m_v[b], acc_sh.at[di_v[b].at[0]], add=True)

    def body(i, _):
        body_one(2 * i, 0)
        body_one(2 * i + 1, 1)
        return 0

    lax.fori_loop(0, n_chunks // 2, body, 0)
    body_one(n_chunks - 1, 0)

    plsc.subcore_barrier()
    for j in range(RT // GC):
        r0 = rows0 + j * GC
        pltpu.sync_copy(acc_sh.at[pl.ds(r0, GC)], m0)
        pltpu.sync_copy(m0, aggp_hbm.at[pl.ds(c * N2 + r0, GC)])


def _cscatter_chunk(chunk, refs):
    (wd_hbm, di_hbm, di_v, wd_v, cacc_v) = refs
    base = chunk * GC
    pltpu.sync_copy(di_hbm.at[pl.ds(base, GC)], di_v)
    pltpu.sync_copy(wd_hbm.at[:, pl.ds(base, GC)], wd_v)
    for k in range(GC // 16):
        sl = pl.ds(k * 16, 16)
        dstv = di_v[sl]
        for cc in range(4):
            val = wd_v[cc, sl]
            plsc.addupdate_scatter(
                cacc_v, [jnp.full((16,), cc, jnp.int32), dstv], val)


@functools.partial(
    pl.kernel,
    out_type=jax.ShapeDtypeStruct((NW * 8, N2), jnp.float32),
    mesh=plsc.VectorSubcoreMesh(**_SC_MESH),
    scratch_types=[
        pltpu.VMEM((GC,), jnp.int32),
        pltpu.VMEM((4, GC), jnp.float32),
        pltpu.VMEM((8, N2), jnp.float32),
    ],
    compiler_params=pltpu.CompilerParams(needs_layout_passes=False),
)
def _sc_cscatter(wd_hbm, di_hbm, caccp_hbm, di_v, wd_v, cacc_v):
    c = lax.axis_index("c")
    s = lax.axis_index("s")
    wid = s * NC + c
    refs = (wd_hbm, di_hbm, di_v, wd_v, cacc_v)

    def zc(j, _):
        for cc in range(8):
            cacc_v[cc, pl.ds(j * 16, 16)] = jnp.zeros((16,), jnp.float32)
        return 0

    lax.fori_loop(0, N2 // 16, zc, 0)

    def body(r, _):
        _cscatter_chunk(r * NW + wid, refs)
        return 0

    lax.fori_loop(0, FULL_ROUNDS, body, 0)

    @pl.when(wid < TAIL)
    def _():
        _cscatter_chunk(FULL_ROUNDS * NW + wid, refs)

    pltpu.sync_copy(cacc_v, caccp_hbm.at[pl.ds(wid * 8, 8)])


def _edge_body(g1_ref, g2_ref, d4_ref, c2_ref, be1_ref, we2_ref,
               be2_ref, wc1_ref, bc1_ref, wc2_ref, m_ref, wd_ref):
    bf = jnp.bfloat16
    d4 = d4_ref[...]
    dsq = (d4 * d4).astype(bf)
    pre = (g1_ref[...] + g2_ref[...] + be1_ref[...]
           + lax.dot_general(dsq, c2_ref[...].astype(bf),
                             (((0,), (0,)), ((), ())),
                             preferred_element_type=jnp.float32))
    m1 = _silu(pre).astype(bf)
    m = _silu(jnp.dot(m1, we2_ref[...].astype(bf),
                      preferred_element_type=jnp.float32) + be2_ref[...])
    m_ref[...] = m
    c = _silu(jnp.dot(m.astype(bf), wc1_ref[...].astype(bf),
                      preferred_element_type=jnp.float32) + bc1_ref[...])
    cw_w = lax.dot_general(wc2_ref[...].astype(bf), c.astype(bf),
                           (((1,), (1,)), ((), ())),
                           preferred_element_type=jnp.float32)
    is3 = lax.broadcasted_iota(jnp.int32, d4.shape, 0) == 3
    wd_ref[...] = jnp.where(is3, 1.0, d4 * cw_w)


def _edge_mlp(g1, g2, d4, C2, b_e1, We2T, b_e2, Wc1T, b_c1, Wc2):
    full = lambda r, c: pl.BlockSpec((r, c), lambda i: (0, 0))
    return pl.pallas_call(
        _edge_body,
        grid=(E // BE,),
        in_specs=[
            pl.BlockSpec((BE, H), lambda i: (i, 0)),
            pl.BlockSpec((BE, H), lambda i: (i, 0)),
            pl.BlockSpec((4, BE), lambda i: (0, i)),
            full(4, H), full(1, H), full(H, H), full(1, H),
            full(H, H), full(1, H), full(1, H),
        ],
        out_specs=[
            pl.BlockSpec((BE, H), lambda i: (i, 0)),
            pl.BlockSpec((4, BE), lambda i: (0, i)),
        ],
        out_shape=[
            jax.ShapeDtypeStruct((E, H), jnp.float32),
            jax.ShapeDtypeStruct((4, E), jnp.float32),
        ],
    )(g1, g2, d4, C2, b_e1, We2T, b_e2, Wc1T, b_c1, Wc2)


def _node_body(h_ref, x4w_ref, aggp0_ref, aggp1_ref, cgp_ref,
               n1a_ref, n1b_ref, bn1_ref, n2_ref, bn2_ref, lnw_ref, lnb_ref,
               ho_ref, xo_ref):
    hb = h_ref[...]
    cg = jnp.sum(cgp_ref[...], axis=0)[:4, :]
    e41 = (lax.broadcasted_iota(jnp.int32, (4, 1), 0) == 3).astype(jnp.float32)
    cnt_col = jnp.maximum(
        lax.dot_general(cg, e41, (((0,), (0,)), ((), ())),
                        preferred_element_type=jnp.float32), 1.0)
    agg = (aggp0_ref[...] + aggp1_ref[...]) / cnt_col
    u = _silu(jnp.dot(hb, n1a_ref[...], preferred_element_type=jnp.float32)
              + jnp.dot(agg, n1b_ref[...], preferred_element_type=jnp.float32)
              + bn1_ref[...])
    t = jnp.dot(u, n2_ref[...], preferred_element_type=jnp.float32) + bn2_ref[...]
    ho = hb + t
    mu = jnp.mean(ho, axis=-1, keepdims=True)
    d = ho - mu
    var = jnp.mean(d * d, axis=-1, keepdims=True)
    ho_ref[...] = d * lax.rsqrt(var + 1e-5) * lnw_ref[...] + lnb_ref[...]
    cnt_w = jnp.maximum(cg[3:4, :], 1.0)
    xo_ref[...] = x4w_ref[...] + cg / cnt_w


def _node_update(h, x4w, aggp, cgp, N1aT, N1bT, b_n1, Wn2T, b_n2,
                 ln_w, ln_b):
    full = lambda r, c: pl.BlockSpec((r, c), lambda i: (0, 0))
    return pl.pallas_call(
        _node_body,
        grid=(N2 // BN2,),
        in_specs=[
            pl.BlockSpec((BN2, H), lambda i: (i, 0)),
            pl.BlockSpec((4, BN2), lambda i: (0, i)),
            pl.BlockSpec((BN2, H), lambda i: (i, 0)),
            pl.BlockSpec((BN2, H), lambda i: (i + N2 // BN2, 0)),
            pl.BlockSpec((NW, 8, BN2), lambda i: (0, 0, i)),
            full(H, H), full(H, H), full(1, H), full(H, H), full(1, H),
            full(1, H), full(1, H),
        ],
        out_specs=[
            pl.BlockSpec((BN2, H), lambda i: (i, 0)),
            pl.BlockSpec((4, BN2), lambda i: (0, i)),
        ],
        out_shape=[
            jax.ShapeDtypeStruct((N, H), jnp.float32),
            jax.ShapeDtypeStruct((4, N2), jnp.float32),
        ],
    )(h, x4w, aggp, aggp, cgp, N1aT, N1bT, b_n1, Wn2T, b_n2, ln_w, ln_b)


def kernel(h, x, edge_index, W_e1, b_e1, W_e2, b_e2, W_n1, b_n1, W_n2, b_n2,
           W_c1, b_c1, W_c2, ln_w, ln_b):
    src = edge_index[0]
    dst = edge_index[1]
    xt = x.T

    At = W_e1[:, :H].T
    Bt = W_e1[:, H:2 * H].T
    wr = W_e1[:, 2 * H].reshape(1, H)
    C2 = jnp.concatenate([jnp.broadcast_to(wr, (3, H)),
                          jnp.zeros((1, H), jnp.float32)], axis=0)

    p, q = _node_prep(h, At, Bt)

    g1, g2, d4 = _sc_gather(p, q, xt.reshape(3 * N), src, dst)

    m, wd = _edge_mlp(g1, g2, d4, C2, b_e1.reshape(1, H), W_e2.T,
                      b_e2.reshape(1, H), W_c1.T, b_c1.reshape(1, H), W_c2)

    aggp = _sc_scatter(m, dst)
    caccp = _sc_cscatter(wd, dst)
    cgp = caccp.reshape(NW, 8, N2)

    x4w = jnp.pad(xt, ((0, 1), (0, N2 - N)))
    ho, xo = _node_update(h, x4w, aggp, cgp,
                          W_n1[:, :H].T, W_n1[:, H:].T, b_n1.reshape(1, H),
                          W_n2.T, b_n2.reshape(1, H), ln_w.reshape(1, H),
                          ln_b.reshape(1, H))
    return (ho, xo[:3, :N].T)
